# Initial kernel scaffold; baseline (speedup 1.0000x reference)
import jax
import jax.numpy as jnp
from jax.experimental import pallas as pl


def kernel(x, edge_index, W, b, gamma, beta, alpha):
    raise NotImplementedError("write your pallas kernel here")


# speedup vs baseline: 14.1977x; 14.1977x over previous
import functools

import jax
import jax.numpy as jnp
from jax import lax
from jax.experimental import pallas as pl
from jax.experimental.pallas import tpu as pltpu
from jax.experimental.pallas import tpu_sc as plsc

N = 10000
D = 128
NC = 2
NS = 16
NW = NC * NS
K = 128
SROWS = 10240
RPT = SROWS // NS
RB = 1000
NB = N // RB

_f32 = jnp.float32
_sc_mesh = plsc.VectorSubcoreMesh(core_axis_name="c", subcore_axis_name="s")


def _make_deg_kernel(ept):
    @functools.partial(
        pl.kernel,
        out_type=jax.ShapeDtypeStruct((NC, SROWS), _f32),
        mesh=_sc_mesh,
        compiler_params=pltpu.CompilerParams(needs_layout_passes=False),
        scratch_types=[
            pltpu.VMEM((ept,), jnp.int32),
            pltpu.VMEM((SROWS,), _f32),
            pltpu.VMEM((NS, RPT), _f32),
            pltpu.VMEM((RPT,), _f32),
            pltpu.VMEM_SHARED((NS, SROWS), _f32),
        ],
    )
    def _deg_kernel(dst_hbm, out_hbm, dst_v, hist_v, red_v, out_v, sdeg):
        cid = lax.axis_index("c")
        sid = lax.axis_index("s")
        w = cid * NS + sid
        pltpu.sync_copy(dst_hbm.at[w], dst_v)
        z16 = jnp.zeros((16,), _f32)
        o16 = jnp.ones((16,), _f32)

        def zb(t, c):
            hist_v[pl.ds(t * 16, 16)] = z16
            return c

        lax.fori_loop(0, SROWS // 16, zb, 0)

        def ab(t, c):
            idx = dst_v[pl.ds(t * 16, 16)]
            plsc.addupdate_scatter(hist_v, [idx], o16)
            return c

        lax.fori_loop(0, ept // 16, ab, 0)
        pltpu.sync_copy(hist_v, sdeg.at[sid])
        plsc.subcore_barrier()
        pltpu.sync_copy(sdeg.at[:, pl.ds(sid * RPT, RPT)], red_v)

        def rb(t, c):
            v = red_v[0, pl.ds(t * 16, 16)]
            for r in range(1, NS):
                v = v + red_v[r, pl.ds(t * 16, 16)]
            out_v[pl.ds(t * 16, 16)] = v
            return c

        lax.fori_loop(0, RPT // 16, rb, 0)
        pltpu.sync_copy(out_v, out_hbm.at[cid, pl.ds(sid * RPT, RPT)])

    return _deg_kernel


def _make_seg_kernel(nch):
    @functools.partial(
        pl.kernel,
        out_type=jax.ShapeDtypeStruct((NC, SROWS, D), _f32),
        mesh=_sc_mesh,
        scratch_types=[
            pltpu.VMEM((K,), jnp.int32),
            pltpu.VMEM((K,), jnp.int32),
            pltpu.VMEM((K,), jnp.int32),
            pltpu.VMEM((K,), jnp.int32),
            pltpu.VMEM((2, K, D), _f32),
            pltpu.VMEM_SHARED((SROWS, D), _f32),
            pltpu.SemaphoreType.DMA,
            pltpu.SemaphoreType.DMA,
            pltpu.SemaphoreType.DMA,
            pltpu.SemaphoreType.DMA,
            pltpu.SemaphoreType.DMA,
            pltpu.SemaphoreType.DMA,
        ],
    )
    def _seg_kernel(y_hbm, src_hbm, dst_hbm, out_hbm,
                    sb0, sb1, db0, db1, rows_v, sacc,
                    ss0, ss1, sd0, sd1, sg0, sg1):
        cid = lax.axis_index("c")
        sid = lax.axis_index("s")
        w = cid * NS + sid
        z16 = jnp.zeros((16,), _f32)

        def zb(t, c):
            r = t // 8
            cc = t - r * 8
            rows_v[0, r, pl.ds(cc * 16, 16)] = z16
            return c

        lax.fori_loop(0, K * (D // 16), zb, 0)
        for k in range(RPT // K):
            pltpu.sync_copy(rows_v.at[0], sacc.at[pl.ds(sid * RPT + k * K, K)])
        plsc.subcore_barrier()

        pltpu.async_copy(src_hbm.at[w, 0], sb0, ss0)
        pltpu.async_copy(src_hbm.at[w, 1], sb1, ss1)
        pltpu.async_copy(dst_hbm.at[w, 0], db0, sd0)
        pltpu.async_copy(dst_hbm.at[w, 1], db1, sd1)
        pltpu.make_async_copy(src_hbm.at[w, 0], sb0, ss0).wait()
        pltpu.async_copy(y_hbm.at[sb0], rows_v.at[0], sg0)

        def body(t, c):
            j0 = 2 * t
            pltpu.make_async_copy(y_hbm.at[sb0], rows_v.at[0], sg0).wait()

            @pl.when(j0 + 2 < nch)
            def _():
                pltpu.async_copy(src_hbm.at[w, j0 + 2], sb0, ss0)

            pltpu.make_async_copy(src_hbm.at[w, 1], sb1, ss1).wait()
            pltpu.async_copy(y_hbm.at[sb1], rows_v.at[1], sg1)
            pltpu.make_async_copy(dst_hbm.at[w, 0], db0, sd0).wait()
            pltpu.sync_copy(rows_v.at[0], sacc.at[db0], add=True)

            @pl.when(j0 + 2 < nch)
            def _():
                pltpu.async_copy(dst_hbm.at[w, j0 + 2], db0, sd0)

            pltpu.make_async_copy(y_hbm.at[sb1], rows_v.at[1], sg1).wait()

            @pl.when(j0 + 3 < nch)
            def _():
                pltpu.async_copy(src_hbm.at[w, j0 + 3], sb1, ss1)

            @pl.when(j0 + 2 < nch)
            def _():
                pltpu.make_async_copy(src_hbm.at[w, 0], sb0, ss0).wait()
                pltpu.async_copy(y_hbm.at[sb0], rows_v.at[0], sg0)

            pltpu.make_async_copy(dst_hbm.at[w, 1], db1, sd1).wait()
            pltpu.sync_copy(rows_v.at[1], sacc.at[db1], add=True)

            @pl.when(j0 + 3 < nch)
            def _():
                pltpu.async_copy(dst_hbm.at[w, j0 + 3], db1, sd1)

            return c

        lax.fori_loop(0, nch // 2, body, 0)
        plsc.subcore_barrier()
        pltpu.sync_copy(sacc.at[pl.ds(sid * RPT, RPT)],
                        out_hbm.at[cid, pl.ds(sid * RPT, RPT)])

    return _seg_kernel


def _prep_body(p_ref, invd_ref, selfw_ref):
    nrow = SROWS // D
    d = p_ref[0:nrow, :] + p_ref[nrow:2 * nrow, :] + 1.0
    invd_ref[...] = lax.rsqrt(d)
    selfw_ref[...] = 1.0 / d


def _scale_body(x_ref, c_ref, y_ref):
    y_ref[...] = x_ref[...] * c_ref[...]


def _mm_body(s0_ref, s1_ref, x_ref, ci_ref, cs_ref, w_ref, b_ref,
             h_ref, m1_ref, m2_ref):
    i = pl.program_id(0)
    agg = ci_ref[...] * (s0_ref[...] + s1_ref[...]) + cs_ref[...] * x_ref[...]
    h = jnp.dot(agg, w_ref[...], preferred_element_type=_f32) + b_ref[...]
    h_ref[...] = h

    @pl.when(i == 0)
    def _():
        m1_ref[...] = jnp.zeros_like(m1_ref)
        m2_ref[...] = jnp.zeros_like(m2_ref)

    m1_ref[0:1, :] += jnp.sum(h, axis=0, keepdims=True)
    m2_ref[0:1, :] += jnp.sum(h * h, axis=0, keepdims=True)


def _norm_body(h_ref, m1_ref, m2_ref, g_ref, be_ref, al_ref, o_ref):
    inv_n = 1.0 / float(N)
    mean = m1_ref[0:1, :] * inv_n
    ex2 = m2_ref[0:1, :] * inv_n
    a = al_ref[...]
    var = ex2 - (2.0 * a - a * a) * mean * mean
    o_ref[...] = (g_ref[...] * (h_ref[...] - a * mean)
                  * lax.rsqrt(var + 1e-5) + be_ref[...])


def kernel(x, edge_index, W, b, gamma, beta, alpha):
    e = edge_index.shape[1]
    ept = -(-e // (NW * 2 * K)) * (2 * K)
    nch = ept // K
    pad_e = NW * ept - e

    src = edge_index[0]
    dst = edge_index[1]
    src_p = jnp.concatenate([src, jnp.zeros((pad_e,), jnp.int32)])
    dst_p = jnp.concatenate([dst, jnp.full((pad_e,), N, jnp.int32)])
    src3 = src_p.reshape(NW, nch, K)
    dst3 = dst_p.reshape(NW, nch, K)
    dst2 = dst_p.reshape(NW, ept)

    degp = _make_deg_kernel(ept)(dst2)

    p2 = degp.reshape(2 * SROWS // D, D)
    invd2d, selfw2d = pl.pallas_call(
        _prep_body,
        out_shape=[jax.ShapeDtypeStruct((SROWS // D, D), _f32),
                   jax.ShapeDtypeStruct((SROWS // D, D), _f32)],
    )(p2)
    invd_col = invd2d.reshape(SROWS, 1)[:N]
    selfw_col = selfw2d.reshape(SROWS, 1)[:N]

    y = pl.pallas_call(
        _scale_body,
        grid=(NB,),
        in_specs=[pl.BlockSpec((RB, D), lambda i: (i, 0)),
                  pl.BlockSpec((RB, 1), lambda i: (i, 0))],
        out_specs=pl.BlockSpec((RB, D), lambda i: (i, 0)),
        out_shape=jax.ShapeDtypeStruct((N, D), _f32),
    )(x, invd_col)

    sacc = _make_seg_kernel(nch)(y, src3, dst3)

    w2 = W
    b2 = b.reshape(1, D)
    h, m1, m2 = pl.pallas_call(
        _mm_body,
        grid=(NB,),
        in_specs=[pl.BlockSpec((RB, D), lambda i: (i, 0)),
                  pl.BlockSpec((RB, D), lambda i: (i, 0)),
                  pl.BlockSpec((RB, D), lambda i: (i, 0)),
                  pl.BlockSpec((RB, 1), lambda i: (i, 0)),
                  pl.BlockSpec((RB, 1), lambda i: (i, 0)),
                  pl.BlockSpec((D, D), lambda i: (0, 0)),
                  pl.BlockSpec((1, D), lambda i: (0, 0))],
        out_specs=[pl.BlockSpec((RB, D), lambda i: (i, 0)),
                   pl.BlockSpec((8, D), lambda i: (0, 0)),
                   pl.BlockSpec((8, D), lambda i: (0, 0))],
        out_shape=[jax.ShapeDtypeStruct((N, D), _f32),
                   jax.ShapeDtypeStruct((8, D), _f32),
                   jax.ShapeDtypeStruct((8, D), _f32)],
    )(sacc[0], sacc[1], x, invd_col, selfw_col, w2, b2)

    out = pl.pallas_call(
        _norm_body,
        grid=(NB,),
        in_specs=[pl.BlockSpec((RB, D), lambda i: (i, 0)),
                  pl.BlockSpec((8, D), lambda i: (0, 0)),
                  pl.BlockSpec((8, D), lambda i: (0, 0)),
                  pl.BlockSpec((1, D), lambda i: (0, 0)),
                  pl.BlockSpec((1, D), lambda i: (0, 0)),
                  pl.BlockSpec((1, D), lambda i: (0, 0))],
        out_specs=pl.BlockSpec((RB, D), lambda i: (i, 0)),
        out_shape=jax.ShapeDtypeStruct((N, D), _f32),
    )(h, m1, m2, gamma.reshape(1, D), beta.reshape(1, D), alpha.reshape(1, D))
    return out


# speedup vs baseline: 35.4678x; 2.4981x over previous
import functools

import jax
import jax.numpy as jnp
from jax import lax
from jax.experimental import pallas as pl
from jax.experimental.pallas import tpu as pltpu
from jax.experimental.pallas import tpu_sc as plsc

N = 10000
D = 128
NC = 2
NS = 16
NW = NC * NS
K = 128
SROWS = 10240
RPT = SROWS // NS
RB = 1000
NB = N // RB

_f32 = jnp.float32
_sc_mesh = plsc.VectorSubcoreMesh(core_axis_name="c", subcore_axis_name="s")


def _make_deg_kernel(ept):
    @functools.partial(
        pl.kernel,
        out_type=jax.ShapeDtypeStruct((NC, SROWS), _f32),
        mesh=_sc_mesh,
        compiler_params=pltpu.CompilerParams(needs_layout_passes=False),
        scratch_types=[
            pltpu.VMEM((ept,), jnp.int32),
            pltpu.VMEM((SROWS,), _f32),
            pltpu.VMEM((NS, RPT), _f32),
            pltpu.VMEM((RPT,), _f32),
            pltpu.VMEM_SHARED((NS, SROWS), _f32),
        ],
    )
    def _deg_kernel(dst_hbm, out_hbm, dst_v, hist_v, red_v, out_v, sdeg):
        cid = lax.axis_index("c")
        sid = lax.axis_index("s")
        w = cid * NS + sid
        pltpu.sync_copy(dst_hbm.at[w], dst_v)
        z16 = jnp.zeros((16,), _f32)
        o16 = jnp.ones((16,), _f32)

        def zb(t, c):
            hist_v[pl.ds(t * 16, 16)] = z16
            return c

        lax.fori_loop(0, SROWS // 16, zb, 0)

        def ab(t, c):
            idx = dst_v[pl.ds(t * 16, 16)]
            plsc.addupdate_scatter(hist_v, [idx], o16)
            return c

        lax.fori_loop(0, ept // 16, ab, 0)
        pltpu.sync_copy(hist_v, sdeg.at[sid])
        plsc.subcore_barrier()
        pltpu.sync_copy(sdeg.at[:, pl.ds(sid * RPT, RPT)], red_v)

        def rb(t, c):
            v = red_v[0, pl.ds(t * 16, 16)]
            for r in range(1, NS):
                v = v + red_v[r, pl.ds(t * 16, 16)]
            out_v[pl.ds(t * 16, 16)] = v
            return c

        lax.fori_loop(0, RPT // 16, rb, 0)
        pltpu.sync_copy(out_v, out_hbm.at[cid, pl.ds(sid * RPT, RPT)])

    return _deg_kernel


def _make_seg_kernel(nch):
    @functools.partial(
        pl.kernel,
        out_type=jax.ShapeDtypeStruct((NC, SROWS, D), _f32),
        mesh=_sc_mesh,
        scratch_types=[
            pltpu.VMEM((K,), jnp.int32),
            pltpu.VMEM((K,), jnp.int32),
            pltpu.VMEM((K,), jnp.int32),
            pltpu.VMEM((K,), jnp.int32),
            pltpu.VMEM((2, K, D), _f32),
            pltpu.VMEM_SHARED((SROWS, D), _f32),
            pltpu.SemaphoreType.DMA,
            pltpu.SemaphoreType.DMA,
            pltpu.SemaphoreType.DMA,
            pltpu.SemaphoreType.DMA,
            pltpu.SemaphoreType.DMA,
            pltpu.SemaphoreType.DMA,
        ],
    )
    def _seg_kernel(y_hbm, src_hbm, dst_hbm, out_hbm,
                    sb0, sb1, db0, db1, rows_v, sacc,
                    ss0, ss1, sd0, sd1, sg0, sg1):
        cid = lax.axis_index("c")
        sid = lax.axis_index("s")
        w = cid * NS + sid
        z16 = jnp.zeros((16,), _f32)

        def zb(t, c):
            r = t // 8
            cc = t - r * 8
            rows_v[0, r, pl.ds(cc * 16, 16)] = z16
            return c

        lax.fori_loop(0, K * (D // 16), zb, 0)
        for k in range(RPT // K):
            pltpu.sync_copy(rows_v.at[0], sacc.at[pl.ds(sid * RPT + k * K, K)])
        plsc.subcore_barrier()

        pltpu.async_copy(src_hbm.at[w, 0], sb0, ss0)
        pltpu.async_copy(src_hbm.at[w, 1], sb1, ss1)
        pltpu.async_copy(dst_hbm.at[w, 0], db0, sd0)
        pltpu.async_copy(dst_hbm.at[w, 1], db1, sd1)
        pltpu.make_async_copy(src_hbm.at[w, 0], sb0, ss0).wait()
        pltpu.async_copy(y_hbm.at[sb0], rows_v.at[0], sg0)

        def body(t, c):
            j0 = 2 * t
            pltpu.make_async_copy(y_hbm.at[sb0], rows_v.at[0], sg0).wait()

            @pl.when(j0 + 2 < nch)
            def _():
                pltpu.async_copy(src_hbm.at[w, j0 + 2], sb0, ss0)

            pltpu.make_async_copy(src_hbm.at[w, 1], sb1, ss1).wait()
            pltpu.async_copy(y_hbm.at[sb1], rows_v.at[1], sg1)
            pltpu.make_async_copy(dst_hbm.at[w, 0], db0, sd0).wait()
            pltpu.sync_copy(rows_v.at[0], sacc.at[db0], add=True)

            @pl.when(j0 + 2 < nch)
            def _():
                pltpu.async_copy(dst_hbm.at[w, j0 + 2], db0, sd0)

            pltpu.make_async_copy(y_hbm.at[sb1], rows_v.at[1], sg1).wait()

            @pl.when(j0 + 3 < nch)
            def _():
                pltpu.async_copy(src_hbm.at[w, j0 + 3], sb1, ss1)

            @pl.when(j0 + 2 < nch)
            def _():
                pltpu.make_async_copy(src_hbm.at[w, 0], sb0, ss0).wait()
                pltpu.async_copy(y_hbm.at[sb0], rows_v.at[0], sg0)

            pltpu.make_async_copy(dst_hbm.at[w, 1], db1, sd1).wait()
            pltpu.sync_copy(rows_v.at[1], sacc.at[db1], add=True)

            @pl.when(j0 + 3 < nch)
            def _():
                pltpu.async_copy(dst_hbm.at[w, j0 + 3], db1, sd1)

            return c

        lax.fori_loop(0, nch // 2, body, 0)
        plsc.subcore_barrier()
        pltpu.sync_copy(sacc.at[pl.ds(sid * RPT, RPT)],
                        out_hbm.at[cid, pl.ds(sid * RPT, RPT)])

    return _seg_kernel


def _prep_body(p_ref, invd_ref, selfw_ref):
    nrow = SROWS // D
    d = p_ref[0:nrow, :] + p_ref[nrow:2 * nrow, :] + 1.0
    invd_ref[...] = lax.rsqrt(d)
    selfw_ref[...] = 1.0 / d


def _scale_body(x_ref, c_ref, y_ref):
    y_ref[...] = x_ref[...] * c_ref[...]


def _mm_body(s0_ref, s1_ref, x_ref, ci_ref, cs_ref, w_ref, b_ref,
             h_ref, m1_ref, m2_ref):
    i = pl.program_id(0)
    agg = ci_ref[...] * (s0_ref[...] + s1_ref[...]) + cs_ref[...] * x_ref[...]
    h = jnp.dot(agg, w_ref[...], preferred_element_type=_f32) + b_ref[...]
    h_ref[...] = h

    @pl.when(i == 0)
    def _():
        m1_ref[...] = jnp.zeros_like(m1_ref)
        m2_ref[...] = jnp.zeros_like(m2_ref)

    m1_ref[0:1, :] += jnp.sum(h, axis=0, keepdims=True)
    m2_ref[0:1, :] += jnp.sum(h * h, axis=0, keepdims=True)


def _norm_body(h_ref, m1_ref, m2_ref, g_ref, be_ref, al_ref, o_ref):
    inv_n = 1.0 / float(N)
    mean = m1_ref[0:1, :] * inv_n
    ex2 = m2_ref[0:1, :] * inv_n
    a = al_ref[...]
    var = ex2 - (2.0 * a - a * a) * mean * mean
    o_ref[...] = (g_ref[...] * (h_ref[...] - a * mean)
                  * lax.rsqrt(var + 1e-5) + be_ref[...])


def kernel(x, edge_index, W, b, gamma, beta, alpha):
    e = edge_index.shape[1]
    ept = -(-e // (NW * 2 * K)) * (2 * K)
    nch = ept // K
    pad_e = NW * ept - e

    src = edge_index[0]
    dst = edge_index[1]
    pad_ar = jnp.arange(pad_e, dtype=jnp.int32)
    src_p = jnp.concatenate([src, pad_ar % N])
    dst_p = jnp.concatenate([dst, N + pad_ar % (SROWS - N)])
    src3 = src_p.reshape(NW, nch, K)
    dst3 = dst_p.reshape(NW, nch, K)
    dst2 = dst_p.reshape(NW, ept)

    degp = _make_deg_kernel(ept)(dst2)

    p2 = degp.reshape(2 * SROWS // D, D)
    invd2d, selfw2d = pl.pallas_call(
        _prep_body,
        out_shape=[jax.ShapeDtypeStruct((SROWS // D, D), _f32),
                   jax.ShapeDtypeStruct((SROWS // D, D), _f32)],
    )(p2)
    invd_col = invd2d.reshape(SROWS, 1)[:N]
    selfw_col = selfw2d.reshape(SROWS, 1)[:N]

    y = pl.pallas_call(
        _scale_body,
        grid=(NB,),
        in_specs=[pl.BlockSpec((RB, D), lambda i: (i, 0)),
                  pl.BlockSpec((RB, 1), lambda i: (i, 0))],
        out_specs=pl.BlockSpec((RB, D), lambda i: (i, 0)),
        out_shape=jax.ShapeDtypeStruct((N, D), _f32),
    )(x, invd_col)

    sacc = _make_seg_kernel(nch)(y, src3, dst3)

    w2 = W
    b2 = b.reshape(1, D)
    h, m1, m2 = pl.pallas_call(
        _mm_body,
        grid=(NB,),
        in_specs=[pl.BlockSpec((RB, D), lambda i: (i, 0)),
                  pl.BlockSpec((RB, D), lambda i: (i, 0)),
                  pl.BlockSpec((RB, D), lambda i: (i, 0)),
                  pl.BlockSpec((RB, 1), lambda i: (i, 0)),
                  pl.BlockSpec((RB, 1), lambda i: (i, 0)),
                  pl.BlockSpec((D, D), lambda i: (0, 0)),
                  pl.BlockSpec((1, D), lambda i: (0, 0))],
        out_specs=[pl.BlockSpec((RB, D), lambda i: (i, 0)),
                   pl.BlockSpec((8, D), lambda i: (0, 0)),
                   pl.BlockSpec((8, D), lambda i: (0, 0))],
        out_shape=[jax.ShapeDtypeStruct((N, D), _f32),
                   jax.ShapeDtypeStruct((8, D), _f32),
                   jax.ShapeDtypeStruct((8, D), _f32)],
    )(sacc[0], sacc[1], x, invd_col, selfw_col, w2, b2)

    out = pl.pallas_call(
        _norm_body,
        grid=(NB,),
        in_specs=[pl.BlockSpec((RB, D), lambda i: (i, 0)),
                  pl.BlockSpec((8, D), lambda i: (0, 0)),
                  pl.BlockSpec((8, D), lambda i: (0, 0)),
                  pl.BlockSpec((1, D), lambda i: (0, 0)),
                  pl.BlockSpec((1, D), lambda i: (0, 0)),
                  pl.BlockSpec((1, D), lambda i: (0, 0))],
        out_specs=pl.BlockSpec((RB, D), lambda i: (i, 0)),
        out_shape=jax.ShapeDtypeStruct((N, D), _f32),
    )(h, m1, m2, gamma.reshape(1, D), beta.reshape(1, D), alpha.reshape(1, D))
    return out


# speedup vs baseline: 36.2593x; 1.0223x over previous
import functools

import jax
import jax.numpy as jnp
from jax import lax
from jax.experimental import pallas as pl
from jax.experimental.pallas import tpu as pltpu
from jax.experimental.pallas import tpu_sc as plsc

N = 10000
D = 128
NC = 2
NS = 16
NW = NC * NS
K = 64
NBUF = 4
NIDX = 8
SROWS = 10240
RPT = SROWS // NS
RB = 1000
NB = N // RB

_f32 = jnp.float32
_sc_mesh = plsc.VectorSubcoreMesh(core_axis_name="c", subcore_axis_name="s")


def _make_deg_kernel(ept):
    @functools.partial(
        pl.kernel,
        out_type=jax.ShapeDtypeStruct((NC, SROWS), _f32),
        mesh=_sc_mesh,
        compiler_params=pltpu.CompilerParams(needs_layout_passes=False),
        scratch_types=[
            pltpu.VMEM((ept,), jnp.int32),
            pltpu.VMEM((SROWS,), _f32),
            pltpu.VMEM((NS, RPT), _f32),
            pltpu.VMEM((RPT,), _f32),
            pltpu.VMEM_SHARED((NS, SROWS), _f32),
        ],
    )
    def _deg_kernel(dst_hbm, out_hbm, dst_v, hist_v, red_v, out_v, sdeg):
        cid = lax.axis_index("c")
        sid = lax.axis_index("s")
        w = cid * NS + sid
        pltpu.sync_copy(dst_hbm.at[w], dst_v)
        z16 = jnp.zeros((16,), _f32)
        o16 = jnp.ones((16,), _f32)

        def zb(t, c):
            hist_v[pl.ds(t * 16, 16)] = z16
            return c

        lax.fori_loop(0, SROWS // 16, zb, 0)

        def ab(t, c):
            idx = dst_v[pl.ds(t * 16, 16)]
            plsc.addupdate_scatter(hist_v, [idx], o16)
            return c

        lax.fori_loop(0, ept // 16, ab, 0)
        pltpu.sync_copy(hist_v, sdeg.at[sid])
        plsc.subcore_barrier()
        pltpu.sync_copy(sdeg.at[:, pl.ds(sid * RPT, RPT)], red_v)

        def rb_(t, c):
            v = red_v[0, pl.ds(t * 16, 16)]
            for r in range(1, NS):
                v = v + red_v[r, pl.ds(t * 16, 16)]
            out_v[pl.ds(t * 16, 16)] = v
            return c

        lax.fori_loop(0, RPT // 16, rb_, 0)
        pltpu.sync_copy(out_v, out_hbm.at[cid, pl.ds(sid * RPT, RPT)])

    return _deg_kernel


def _make_seg_kernel(nch):
    @functools.partial(
        pl.kernel,
        out_type=jax.ShapeDtypeStruct((NC, SROWS, D), _f32),
        mesh=_sc_mesh,
        scratch_types=[
            pltpu.VMEM((NIDX, K), jnp.int32),
            pltpu.VMEM((NIDX, K), jnp.int32),
            pltpu.VMEM((NBUF, K, D), _f32),
            pltpu.VMEM_SHARED((SROWS, D), _f32),
            pltpu.SemaphoreType.DMA((NIDX,)),
            pltpu.SemaphoreType.DMA((NIDX,)),
            pltpu.SemaphoreType.DMA((NBUF,)),
            pltpu.SemaphoreType.DMA((NBUF,)),
        ],
    )
    def _seg_kernel(y_hbm, src_hbm, dst_hbm, out_hbm,
                    sbuf, dbuf, rbuf, sacc, si, sj, sg, sc):
        cid = lax.axis_index("c")
        sid = lax.axis_index("s")
        w = cid * NS + sid
        z16 = jnp.zeros((16,), _f32)

        def zb(t, c):
            r = t // 8
            cc = t - r * 8
            rbuf[0, r, pl.ds(cc * 16, 16)] = z16
            return c

        lax.fori_loop(0, K * (D // 16), zb, 0)
        for k in range(RPT // K):
            pltpu.sync_copy(rbuf.at[0], sacc.at[pl.ds(sid * RPT + k * K, K)])
        plsc.subcore_barrier()

        def idx_issue(j, q):
            pltpu.async_copy(src_hbm.at[w, j], sbuf.at[q], si.at[q])
            pltpu.async_copy(dst_hbm.at[w, j], dbuf.at[q], sj.at[q])

        def gather_issue(j, q, b):
            pltpu.async_copy(y_hbm.at[sbuf.at[q]], rbuf.at[b], sg.at[b])

        def gather_wait(q, b):
            pltpu.make_async_copy(y_hbm.at[sbuf.at[q]], rbuf.at[b],
                                  sg.at[b]).wait()

        def scatter_issue(q, b):
            pltpu.async_copy(rbuf.at[b], sacc.at[dbuf.at[q]], sc.at[b],
                             add=True)

        def scatter_wait(q, b):
            pltpu.make_async_copy(rbuf.at[b], sacc.at[dbuf.at[q]],
                                  sc.at[b]).wait()

        for q in range(6):
            idx_issue(q, q)
        for b in range(2):
            pltpu.make_async_copy(src_hbm.at[w, b], sbuf.at[b],
                                  si.at[b]).wait()
            gather_issue(b, b, b)

        def body(t, c):
            for k in range(NIDX):
                j = NIDX * t + k
                b = k % NBUF
                q2 = (k + 2) % NIDX
                b2 = (k + 2) % NBUF
                q6 = (k + 6) % NIDX
                gather_wait(k, b)
                pltpu.make_async_copy(dst_hbm.at[w, j], dbuf.at[k],
                                      sj.at[k]).wait()
                scatter_issue(k, b)

                @pl.when(jnp.logical_and(j + 2 < nch, j >= 2))
                def _():
                    scatter_wait(q2, b2)

                @pl.when(j + 2 < nch)
                def _():
                    pltpu.make_async_copy(src_hbm.at[w, j], sbuf.at[q2],
                                          si.at[q2]).wait()
                    gather_issue(j + 2, q2, b2)

                @pl.when(j + 6 < nch)
                def _():
                    idx_issue(j + 6, q6)
            return c

        lax.fori_loop(0, nch // NIDX, body, 0)
        for b in range(NBUF):
            scatter_wait(0, b)
        plsc.subcore_barrier()
        pltpu.sync_copy(sacc.at[pl.ds(sid * RPT, RPT)],
                        out_hbm.at[cid, pl.ds(sid * RPT, RPT)])

    return _seg_kernel


def _scale_body(x_ref, p0_ref, p1_ref, y_ref, invd_ref, selfw_ref):
    d = p0_ref[...] + p1_ref[...] + 1.0
    invd = lax.rsqrt(d)
    invd_ref[...] = invd
    selfw_ref[...] = 1.0 / d
    y_ref[...] = x_ref[...] * invd


def _mm_body(s0_ref, s1_ref, x_ref, ci_ref, cs_ref, w_ref, b_ref,
             h_ref, m1_ref, m2_ref):
    i = pl.program_id(0)
    agg = ci_ref[...] * (s0_ref[...] + s1_ref[...]) + cs_ref[...] * x_ref[...]
    h = jnp.dot(agg, w_ref[...], preferred_element_type=_f32) + b_ref[...]
    h_ref[...] = h

    @pl.when(i == 0)
    def _():
        m1_ref[...] = jnp.zeros_like(m1_ref)
        m2_ref[...] = jnp.zeros_like(m2_ref)

    m1_ref[0:1, :] += jnp.sum(h, axis=0, keepdims=True)
    m2_ref[0:1, :] += jnp.sum(h * h, axis=0, keepdims=True)


def _norm_body(h_ref, m1_ref, m2_ref, g_ref, be_ref, al_ref, o_ref):
    inv_n = 1.0 / float(N)
    mean = m1_ref[0:1, :] * inv_n
    ex2 = m2_ref[0:1, :] * inv_n
    a = al_ref[...]
    var = ex2 - (2.0 * a - a * a) * mean * mean
    o_ref[...] = (g_ref[...] * (h_ref[...] - a * mean)
                  * lax.rsqrt(var + 1e-5) + be_ref[...])


def kernel(x, edge_index, W, b, gamma, beta, alpha):
    e = edge_index.shape[1]
    ept = -(-e // (NW * NIDX * K)) * (NIDX * K)
    nch = ept // K
    pad_e = NW * ept - e

    src = edge_index[0]
    dst = edge_index[1]
    pad_ar = jnp.arange(pad_e, dtype=jnp.int32)
    src_p = jnp.concatenate([src, pad_ar % N])
    dst_p = jnp.concatenate([dst, N + pad_ar % (SROWS - N)])
    src3 = src_p.reshape(NW, nch, K)
    dst3 = dst_p.reshape(NW, nch, K)
    dst2 = dst_p.reshape(NW, ept)

    degp = _make_deg_kernel(ept)(dst2)

    dp = degp.reshape(NC, SROWS, 1)
    y, invd_col, selfw_col = pl.pallas_call(
        _scale_body,
        grid=(NB,),
        in_specs=[pl.BlockSpec((RB, D), lambda i: (i, 0)),
                  pl.BlockSpec((RB, 1), lambda i: (i, 0)),
                  pl.BlockSpec((RB, 1), lambda i: (i, 0))],
        out_specs=[pl.BlockSpec((RB, D), lambda i: (i, 0)),
                   pl.BlockSpec((RB, 1), lambda i: (i, 0)),
                   pl.BlockSpec((RB, 1), lambda i: (i, 0))],
        out_shape=[jax.ShapeDtypeStruct((N, D), _f32),
                   jax.ShapeDtypeStruct((N, 1), _f32),
                   jax.ShapeDtypeStruct((N, 1), _f32)],
    )(x, dp[0], dp[1])

    sacc = _make_seg_kernel(nch)(y, src3, dst3)

    b2 = b.reshape(1, D)
    h, m1, m2 = pl.pallas_call(
        _mm_body,
        grid=(NB,),
        in_specs=[pl.BlockSpec((RB, D), lambda i: (i, 0)),
                  pl.BlockSpec((RB, D), lambda i: (i, 0)),
                  pl.BlockSpec((RB, D), lambda i: (i, 0)),
                  pl.BlockSpec((RB, 1), lambda i: (i, 0)),
                  pl.BlockSpec((RB, 1), lambda i: (i, 0)),
                  pl.BlockSpec((D, D), lambda i: (0, 0)),
                  pl.BlockSpec((1, D), lambda i: (0, 0))],
        out_specs=[pl.BlockSpec((RB, D), lambda i: (i, 0)),
                   pl.BlockSpec((8, D), lambda i: (0, 0)),
                   pl.BlockSpec((8, D), lambda i: (0, 0))],
        out_shape=[jax.ShapeDtypeStruct((N, D), _f32),
                   jax.ShapeDtypeStruct((8, D), _f32),
                   jax.ShapeDtypeStruct((8, D), _f32)],
    )(sacc[0], sacc[1], x, invd_col, selfw_col, W, b2)

    out = pl.pallas_call(
        _norm_body,
        grid=(NB,),
        in_specs=[pl.BlockSpec((RB, D), lambda i: (i, 0)),
                  pl.BlockSpec((8, D), lambda i: (0, 0)),
                  pl.BlockSpec((8, D), lambda i: (0, 0)),
                  pl.BlockSpec((1, D), lambda i: (0, 0)),
                  pl.BlockSpec((1, D), lambda i: (0, 0)),
                  pl.BlockSpec((1, D), lambda i: (0, 0))],
        out_specs=pl.BlockSpec((RB, D), lambda i: (i, 0)),
        out_shape=jax.ShapeDtypeStruct((N, D), _f32),
    )(h, m1, m2, gamma.reshape(1, D), beta.reshape(1, D), alpha.reshape(1, D))
    return out


# speedup vs baseline: 37.2173x; 1.0264x over previous
import functools

import jax
import jax.numpy as jnp
from jax import lax
from jax.experimental import pallas as pl
from jax.experimental.pallas import tpu as pltpu
from jax.experimental.pallas import tpu_sc as plsc

N = 10000
D = 128
NC = 2
NS = 16
NW = NC * NS
K = 64
NBUF = 4
NIDX = 8
SROWS = 10240
RPT = SROWS // NS
RB = 1000
NB = N // RB

_f32 = jnp.float32
_sc_mesh = plsc.VectorSubcoreMesh(core_axis_name="c", subcore_axis_name="s")


def _make_deg_kernel(ept):
    @functools.partial(
        pl.kernel,
        out_type=jax.ShapeDtypeStruct((NC, SROWS), _f32),
        mesh=_sc_mesh,
        compiler_params=pltpu.CompilerParams(needs_layout_passes=False),
        scratch_types=[
            pltpu.VMEM((ept,), jnp.int32),
            pltpu.VMEM((SROWS,), _f32),
            pltpu.VMEM((NS, RPT), _f32),
            pltpu.VMEM((RPT,), _f32),
            pltpu.VMEM_SHARED((NS, SROWS), _f32),
        ],
    )
    def _deg_kernel(dst_hbm, out_hbm, dst_v, hist_v, red_v, out_v, sdeg):
        cid = lax.axis_index("c")
        sid = lax.axis_index("s")
        w = cid * NS + sid
        pltpu.sync_copy(dst_hbm.at[w], dst_v)
        z16 = jnp.zeros((16,), _f32)
        o16 = jnp.ones((16,), _f32)

        def zb(t, c):
            hist_v[pl.ds(t * 16, 16)] = z16
            return c

        lax.fori_loop(0, SROWS // 16, zb, 0)

        def ab(t, c):
            idx = dst_v[pl.ds(t * 16, 16)]
            plsc.addupdate_scatter(hist_v, [idx], o16)
            return c

        lax.fori_loop(0, ept // 16, ab, 0)
        pltpu.sync_copy(hist_v, sdeg.at[sid])
        plsc.subcore_barrier()
        pltpu.sync_copy(sdeg.at[:, pl.ds(sid * RPT, RPT)], red_v)

        def rb_(t, c):
            v = red_v[0, pl.ds(t * 16, 16)]
            for r in range(1, NS):
                v = v + red_v[r, pl.ds(t * 16, 16)]
            out_v[pl.ds(t * 16, 16)] = v
            return c

        lax.fori_loop(0, RPT // 16, rb_, 0)
        pltpu.sync_copy(out_v, out_hbm.at[cid, pl.ds(sid * RPT, RPT)])

    return _deg_kernel


def _make_seg_kernel(nch):
    @functools.partial(
        pl.kernel,
        out_type=[jax.ShapeDtypeStruct((SROWS, D), _f32),
                  jax.ShapeDtypeStruct((SROWS, D), _f32)],
        mesh=_sc_mesh,
        scratch_types=[
            pltpu.VMEM((NIDX, K), jnp.int32),
            pltpu.VMEM((NIDX, K), jnp.int32),
            pltpu.VMEM((NBUF, K, D), _f32),
            pltpu.VMEM_SHARED((SROWS, D), _f32),
            pltpu.SemaphoreType.DMA((NIDX,)),
            pltpu.SemaphoreType.DMA((NIDX,)),
            pltpu.SemaphoreType.DMA((NBUF,)),
            pltpu.SemaphoreType.DMA((NBUF,)),
        ],
    )
    def _seg_kernel(y_hbm, src_hbm, dst_hbm, out0_hbm, out1_hbm,
                    sbuf, dbuf, rbuf, sacc, si, sj, sg, sc):
        cid = lax.axis_index("c")
        sid = lax.axis_index("s")
        w = cid * NS + sid
        z16 = jnp.zeros((16,), _f32)

        def zb(t, c):
            r = t // 8
            cc = t - r * 8
            rbuf[0, r, pl.ds(cc * 16, 16)] = z16
            return c

        lax.fori_loop(0, K * (D // 16), zb, 0)
        for k in range(RPT // K):
            pltpu.sync_copy(rbuf.at[0], sacc.at[pl.ds(sid * RPT + k * K, K)])
        plsc.subcore_barrier()

        def idx_issue(j, q):
            pltpu.async_copy(src_hbm.at[w, j], sbuf.at[q], si.at[q])
            pltpu.async_copy(dst_hbm.at[w, j], dbuf.at[q], sj.at[q])

        def gather_issue(j, q, b):
            pltpu.async_copy(y_hbm.at[sbuf.at[q]], rbuf.at[b], sg.at[b])

        def gather_wait(q, b):
            pltpu.make_async_copy(y_hbm.at[sbuf.at[q]], rbuf.at[b],
                                  sg.at[b]).wait()

        def scatter_issue(q, b):
            pltpu.async_copy(rbuf.at[b], sacc.at[dbuf.at[q]], sc.at[b],
                             add=True)

        def scatter_wait(q, b):
            pltpu.make_async_copy(rbuf.at[b], sacc.at[dbuf.at[q]],
                                  sc.at[b]).wait()

        for q in range(6):
            idx_issue(q, q)
        for b in range(2):
            pltpu.make_async_copy(src_hbm.at[w, b], sbuf.at[b],
                                  si.at[b]).wait()
            gather_issue(b, b, b)

        def body(t, c):
            for k in range(NIDX):
                j = NIDX * t + k
                b = k % NBUF
                q2 = (k + 2) % NIDX
                b2 = (k + 2) % NBUF
                q6 = (k + 6) % NIDX
                gather_wait(k, b)
                pltpu.make_async_copy(dst_hbm.at[w, j], dbuf.at[k],
                                      sj.at[k]).wait()
                scatter_issue(k, b)

                @pl.when(jnp.logical_and(j + 2 < nch, j >= 2))
                def _():
                    scatter_wait(q2, b2)

                @pl.when(j + 2 < nch)
                def _():
                    pltpu.make_async_copy(src_hbm.at[w, j], sbuf.at[q2],
                                          si.at[q2]).wait()
                    gather_issue(j + 2, q2, b2)

                @pl.when(j + 6 < nch)
                def _():
                    idx_issue(j + 6, q6)
            return c

        lax.fori_loop(0, nch // NIDX, body, 0)
        for b in range(NBUF):
            scatter_wait(0, b)
        plsc.subcore_barrier()

        @pl.when(cid == 0)
        def _():
            pltpu.sync_copy(sacc.at[pl.ds(sid * RPT, RPT)],
                            out0_hbm.at[pl.ds(sid * RPT, RPT)])

        @pl.when(cid == 1)
        def _():
            pltpu.sync_copy(sacc.at[pl.ds(sid * RPT, RPT)],
                            out1_hbm.at[pl.ds(sid * RPT, RPT)])

    return _seg_kernel


def _scale_body(x_ref, p0_ref, p1_ref, y_ref, invd_ref, selfw_ref):
    d = p0_ref[...] + p1_ref[...] + 1.0
    invd = lax.rsqrt(d)
    invd_ref[...] = invd
    selfw_ref[...] = 1.0 / d
    y_ref[...] = x_ref[...] * invd


def _mmn_body(s0_ref, s1_ref, x_ref, ci_ref, cs_ref, w_ref, b_ref,
              g_ref, be_ref, al_ref, out_ref, h_ref, m1_ref, m2_ref):
    i = pl.program_id(0)

    @pl.when(i < NB)
    def _():
        agg = (ci_ref[...] * (s0_ref[...] + s1_ref[...])
               + cs_ref[...] * x_ref[...])
        h = jnp.dot(agg, w_ref[...], preferred_element_type=_f32) + b_ref[...]
        h_ref[...] = h

        @pl.when(i == 0)
        def _():
            m1_ref[...] = jnp.zeros_like(m1_ref)
            m2_ref[...] = jnp.zeros_like(m2_ref)

        m1_ref[0:1, :] += jnp.sum(h, axis=0, keepdims=True)
        m2_ref[0:1, :] += jnp.sum(h * h, axis=0, keepdims=True)

    @pl.when(i >= NB)
    def _():
        inv_n = 1.0 / float(N)
        mean = m1_ref[0:1, :] * inv_n
        ex2 = m2_ref[0:1, :] * inv_n
        a = al_ref[...]
        var = ex2 - (2.0 * a - a * a) * mean * mean
        h = s0_ref[...]
        h_ref[...] = h
        out_ref[...] = (g_ref[...] * (h - a * mean)
                        * lax.rsqrt(var + 1e-5) + be_ref[...])


def kernel(x, edge_index, W, b, gamma, beta, alpha):
    e = edge_index.shape[1]
    ept = -(-e // (NW * NIDX * K)) * (NIDX * K)
    nch = ept // K
    pad_e = NW * ept - e

    src = edge_index[0]
    dst = edge_index[1]
    pad_ar = jnp.arange(pad_e, dtype=jnp.int32)
    src_p = jnp.concatenate([src, pad_ar % N])
    dst_p = jnp.concatenate([dst, N + pad_ar % (SROWS - N)])
    src3 = src_p.reshape(NW, nch, K)
    dst3 = dst_p.reshape(NW, nch, K)

    if e % (NW * 16) == 0:
        degp = _make_deg_kernel(e // NW)(dst.reshape(NW, e // NW))
    else:
        degp = _make_deg_kernel(ept)(dst_p.reshape(NW, ept))

    dp = degp.reshape(NC, SROWS, 1)
    y, invd_col, selfw_col = pl.pallas_call(
        _scale_body,
        grid=(NB,),
        in_specs=[pl.BlockSpec((RB, D), lambda i: (i, 0)),
                  pl.BlockSpec((RB, 1), lambda i: (i, 0)),
                  pl.BlockSpec((RB, 1), lambda i: (i, 0))],
        out_specs=[pl.BlockSpec((RB, D), lambda i: (i, 0)),
                   pl.BlockSpec((RB, 1), lambda i: (i, 0)),
                   pl.BlockSpec((RB, 1), lambda i: (i, 0))],
        out_shape=[jax.ShapeDtypeStruct((N, D), _f32),
                   jax.ShapeDtypeStruct((N, 1), _f32),
                   jax.ShapeDtypeStruct((N, 1), _f32)],
    )(x, dp[0], dp[1])

    s0a, s1a = _make_seg_kernel(nch)(y, src3, dst3)

    mod_map = lambda i: (i % NB, 0)
    min_map = lambda i: (jnp.minimum(i, NB - 1), 0)
    zero_map = lambda i: (0, 0)
    out, _h, _m1, _m2 = pl.pallas_call(
        _mmn_body,
        grid=(2 * NB,),
        in_specs=[pl.BlockSpec((RB, D), mod_map),
                  pl.BlockSpec((RB, D), min_map),
                  pl.BlockSpec((RB, D), min_map),
                  pl.BlockSpec((RB, 1), min_map),
                  pl.BlockSpec((RB, 1), min_map),
                  pl.BlockSpec((D, D), zero_map),
                  pl.BlockSpec((1, D), zero_map),
                  pl.BlockSpec((1, D), zero_map),
                  pl.BlockSpec((1, D), zero_map),
                  pl.BlockSpec((1, D), zero_map)],
        out_specs=[pl.BlockSpec((RB, D), lambda i: (jnp.maximum(i - NB, 0), 0)),
                   pl.BlockSpec((RB, D), mod_map),
                   pl.BlockSpec((8, D), zero_map),
                   pl.BlockSpec((8, D), zero_map)],
        out_shape=[jax.ShapeDtypeStruct((N, D), _f32),
                   jax.ShapeDtypeStruct((SROWS, D), _f32),
                   jax.ShapeDtypeStruct((8, D), _f32),
                   jax.ShapeDtypeStruct((8, D), _f32)],
        input_output_aliases={0: 1},
    )(s0a, s1a, x, invd_col, selfw_col, W, b.reshape(1, D),
      gamma.reshape(1, D), beta.reshape(1, D), alpha.reshape(1, D))
    return out


# speedup vs baseline: 39.1048x; 1.0507x over previous
import functools

import jax
import jax.numpy as jnp
from jax import lax
from jax.experimental import pallas as pl
from jax.experimental.pallas import tpu as pltpu
from jax.experimental.pallas import tpu_sc as plsc

N = 10000
D = 128
NC = 2
NS = 16
NW = NC * NS
K = 64
NBUF = 4
NIDX = 8
SROWS = 10240
RPT = SROWS // NS
RB = 2000
NB = N // RB

_f32 = jnp.float32
_sc_mesh = plsc.VectorSubcoreMesh(core_axis_name="c", subcore_axis_name="s")


def _make_deg_kernel(ept):
    @functools.partial(
        pl.kernel,
        out_type=jax.ShapeDtypeStruct((NC, SROWS), _f32),
        mesh=_sc_mesh,
        compiler_params=pltpu.CompilerParams(needs_layout_passes=False),
        scratch_types=[
            pltpu.VMEM((ept,), jnp.int32),
            pltpu.VMEM((SROWS,), _f32),
            pltpu.VMEM((NS, RPT), _f32),
            pltpu.VMEM((RPT,), _f32),
            pltpu.VMEM_SHARED((NS, SROWS), _f32),
        ],
    )
    def _deg_kernel(dst_hbm, out_hbm, dst_v, hist_v, red_v, out_v, sdeg):
        cid = lax.axis_index("c")
        sid = lax.axis_index("s")
        w = cid * NS + sid
        pltpu.sync_copy(dst_hbm.at[w], dst_v)
        z16 = jnp.zeros((16,), _f32)
        o16 = jnp.ones((16,), _f32)

        def zb(t, c):
            hist_v[pl.ds(t * 16, 16)] = z16
            return c

        lax.fori_loop(0, SROWS // 16, zb, 0)

        def ab(t, c):
            idx = dst_v[pl.ds(t * 16, 16)]
            plsc.addupdate_scatter(hist_v, [idx], o16)
            return c

        lax.fori_loop(0, ept // 16, ab, 0)
        pltpu.sync_copy(hist_v, sdeg.at[sid])
        plsc.subcore_barrier()
        pltpu.sync_copy(sdeg.at[:, pl.ds(sid * RPT, RPT)], red_v)

        def rb_(t, c):
            v = red_v[0, pl.ds(t * 16, 16)]
            for r in range(1, NS):
                v = v + red_v[r, pl.ds(t * 16, 16)]
            out_v[pl.ds(t * 16, 16)] = v
            return c

        lax.fori_loop(0, RPT // 16, rb_, 0)
        pltpu.sync_copy(out_v, out_hbm.at[cid, pl.ds(sid * RPT, RPT)])

    return _deg_kernel


def _make_seg_kernel(nch, tk, ept):
    @functools.partial(
        pl.kernel,
        out_type=[jax.ShapeDtypeStruct((SROWS, D), _f32),
                  jax.ShapeDtypeStruct((SROWS, D), _f32)],
        mesh=_sc_mesh,
        scratch_types=[
            pltpu.VMEM((NIDX, K), jnp.int32),
            pltpu.VMEM((NIDX, K), jnp.int32),
            pltpu.VMEM((NBUF, K, D), _f32),
            pltpu.VMEM((max(tk, 8),), jnp.int32),
            pltpu.VMEM((max(tk, 8),), jnp.int32),
            pltpu.VMEM((max(tk, 8), D), _f32),
            pltpu.VMEM_SHARED((SROWS, D), _f32),
            pltpu.SemaphoreType.DMA((NIDX,)),
            pltpu.SemaphoreType.DMA((NIDX,)),
            pltpu.SemaphoreType.DMA((NBUF,)),
            pltpu.SemaphoreType.DMA((NBUF,)),
        ],
    )
    def _seg_kernel(y_hbm, src_hbm, dst_hbm, out0_hbm, out1_hbm,
                    sbuf, dbuf, rbuf, tsb, tdb, trb, sacc, si, sj, sg, sc):
        cid = lax.axis_index("c")
        sid = lax.axis_index("s")
        w = cid * NS + sid
        base = w * ept
        z16 = jnp.zeros((16,), _f32)

        def zb(t, c):
            r = t // 8
            cc = t - r * 8
            rbuf[0, r, pl.ds(cc * 16, 16)] = z16
            return c

        lax.fori_loop(0, K * (D // 16), zb, 0)
        for k in range(RPT // K):
            pltpu.sync_copy(rbuf.at[0], sacc.at[pl.ds(sid * RPT + k * K, K)])
        plsc.subcore_barrier()

        def idx_issue(j, q):
            pltpu.async_copy(src_hbm.at[pl.ds(base + j * K, K)],
                             sbuf.at[q], si.at[q])
            pltpu.async_copy(dst_hbm.at[pl.ds(base + j * K, K)],
                             dbuf.at[q], sj.at[q])

        def gather_issue(j, q, b):
            pltpu.async_copy(y_hbm.at[sbuf.at[q]], rbuf.at[b], sg.at[b])

        def gather_wait(q, b):
            pltpu.make_async_copy(y_hbm.at[sbuf.at[q]], rbuf.at[b],
                                  sg.at[b]).wait()

        def scatter_issue(q, b):
            pltpu.async_copy(rbuf.at[b], sacc.at[dbuf.at[q]], sc.at[b],
                             add=True)

        def scatter_wait(q, b):
            pltpu.make_async_copy(rbuf.at[b], sacc.at[dbuf.at[q]],
                                  sc.at[b]).wait()

        def idx_wait(q):
            pltpu.make_async_copy(src_hbm.at[pl.ds(base, K)], sbuf.at[q],
                                  si.at[q]).wait()

        def didx_wait(q):
            pltpu.make_async_copy(dst_hbm.at[pl.ds(base, K)], dbuf.at[q],
                                  sj.at[q]).wait()

        for q in range(6):
            idx_issue(q, q)
        for b in range(2):
            idx_wait(b)
            gather_issue(b, b, b)

        def body(t, c):
            for k in range(NIDX):
                j = NIDX * t + k
                b = k % NBUF
                q2 = (k + 2) % NIDX
                b2 = (k + 2) % NBUF
                q6 = (k + 6) % NIDX

                @pl.when(j < nch)
                def _():
                    gather_wait(k, b)
                    didx_wait(k)
                    scatter_issue(k, b)

                    @pl.when(jnp.logical_and(j + 2 < nch, j >= 2))
                    def _():
                        scatter_wait(q2, b2)

                    @pl.when(j + 2 < nch)
                    def _():
                        idx_wait(q2)
                        gather_issue(j + 2, q2, b2)

                    @pl.when(j + 6 < nch)
                    def _():
                        idx_issue(j + 6, q6)
            return c

        lax.fori_loop(0, -(-nch // NIDX), body, 0)
        for b in range(NBUF):
            scatter_wait(0, b)
        if tk:
            pltpu.async_copy(src_hbm.at[pl.ds(base + nch * K, tk)], tsb, si.at[0])
            pltpu.async_copy(dst_hbm.at[pl.ds(base + nch * K, tk)], tdb, sj.at[0])
            pltpu.make_async_copy(src_hbm.at[pl.ds(base, tk)], tsb,
                                  si.at[0]).wait()
            pltpu.async_copy(y_hbm.at[tsb], trb, sg.at[0]).wait()
            pltpu.make_async_copy(dst_hbm.at[pl.ds(base, tk)], tdb,
                                  sj.at[0]).wait()
            pltpu.sync_copy(trb, sacc.at[tdb], add=True)
        plsc.subcore_barrier()

        @pl.when(cid == 0)
        def _():
            pltpu.sync_copy(sacc.at[pl.ds(sid * RPT, RPT)],
                            out0_hbm.at[pl.ds(sid * RPT, RPT)])

        @pl.when(cid == 1)
        def _():
            pltpu.sync_copy(sacc.at[pl.ds(sid * RPT, RPT)],
                            out1_hbm.at[pl.ds(sid * RPT, RPT)])

    return _seg_kernel


def _scale_body(x_ref, p0_ref, p1_ref, y_ref, invd_ref, selfw_ref):
    d = p0_ref[...] + p1_ref[...] + 1.0
    invd = lax.rsqrt(d)
    invd_ref[...] = invd
    selfw_ref[...] = 1.0 / d
    y_ref[...] = x_ref[...] * invd


def _mmn_body(s0_ref, s1_ref, x_ref, ci_ref, cs_ref, w_ref, b_ref,
              g_ref, be_ref, al_ref, out_ref, h_ref, m1_ref, m2_ref):
    i = pl.program_id(0)

    @pl.when(i < NB)
    def _():
        agg = (ci_ref[...] * (s0_ref[...] + s1_ref[...])
               + cs_ref[...] * x_ref[...])
        h = jnp.dot(agg, w_ref[...], preferred_element_type=_f32) + b_ref[...]
        h_ref[...] = h

        @pl.when(i == 0)
        def _():
            m1_ref[...] = jnp.zeros_like(m1_ref)
            m2_ref[...] = jnp.zeros_like(m2_ref)

        m1_ref[0:1, :] += jnp.sum(h, axis=0, keepdims=True)
        m2_ref[0:1, :] += jnp.sum(h * h, axis=0, keepdims=True)

    @pl.when(i >= NB)
    def _():
        inv_n = 1.0 / float(N)
        mean = m1_ref[0:1, :] * inv_n
        ex2 = m2_ref[0:1, :] * inv_n
        a = al_ref[...]
        var = ex2 - (2.0 * a - a * a) * mean * mean
        h = s0_ref[...]
        h_ref[...] = h
        out_ref[...] = (g_ref[...] * (h - a * mean)
                        * lax.rsqrt(var + 1e-5) + be_ref[...])


def kernel(x, edge_index, W, b, gamma, beta, alpha):
    e = edge_index.shape[1]
    src = edge_index[0]
    dst = edge_index[1]
    if e % (NW * 16) == 0:
        ept = e // NW
    else:
        ept = -(-e // (NW * 16)) * 16
        pad_ar = jnp.arange(NW * ept - e, dtype=jnp.int32)
        src = jnp.concatenate([src, pad_ar % N])
        dst = jnp.concatenate([dst, N + pad_ar % (SROWS - N)])
    nch = ept // K
    tk = ept - nch * K

    degp = _make_deg_kernel(ept)(dst.reshape(NW, ept))

    dp = degp.reshape(NC, SROWS, 1)
    y, invd_col, selfw_col = pl.pallas_call(
        _scale_body,
        grid=(NB,),
        in_specs=[pl.BlockSpec((RB, D), lambda i: (i, 0)),
                  pl.BlockSpec((RB, 1), lambda i: (i, 0)),
                  pl.BlockSpec((RB, 1), lambda i: (i, 0))],
        out_specs=[pl.BlockSpec((RB, D), lambda i: (i, 0)),
                   pl.BlockSpec((RB, 1), lambda i: (i, 0)),
                   pl.BlockSpec((RB, 1), lambda i: (i, 0))],
        out_shape=[jax.ShapeDtypeStruct((N, D), _f32),
                   jax.ShapeDtypeStruct((N, 1), _f32),
                   jax.ShapeDtypeStruct((N, 1), _f32)],
    )(x, dp[0], dp[1])

    s0a, s1a = _make_seg_kernel(nch, tk, ept)(y, src, dst)

    mod_map = lambda i: (i % NB, 0)
    min_map = lambda i: (jnp.minimum(i, NB - 1), 0)
    zero_map = lambda i: (0, 0)
    out, _h, _m1, _m2 = pl.pallas_call(
        _mmn_body,
        grid=(2 * NB,),
        in_specs=[pl.BlockSpec((RB, D), mod_map),
                  pl.BlockSpec((RB, D), min_map),
                  pl.BlockSpec((RB, D), min_map),
                  pl.BlockSpec((RB, 1), min_map),
                  pl.BlockSpec((RB, 1), min_map),
                  pl.BlockSpec((D, D), zero_map),
                  pl.BlockSpec((1, D), zero_map),
                  pl.BlockSpec((1, D), zero_map),
                  pl.BlockSpec((1, D), zero_map),
                  pl.BlockSpec((1, D), zero_map)],
        out_specs=[pl.BlockSpec((RB, D), lambda i: (jnp.maximum(i - NB, 0), 0)),
                   pl.BlockSpec((RB, D), mod_map),
                   pl.BlockSpec((8, D), zero_map),
                   pl.BlockSpec((8, D), zero_map)],
        out_shape=[jax.ShapeDtypeStruct((N, D), _f32),
                   jax.ShapeDtypeStruct((SROWS, D), _f32),
                   jax.ShapeDtypeStruct((8, D), _f32),
                   jax.ShapeDtypeStruct((8, D), _f32)],
        input_output_aliases={0: 1},
    )(s0a, s1a, x, invd_col, selfw_col, W, b.reshape(1, D),
      gamma.reshape(1, D), beta.reshape(1, D), alpha.reshape(1, D))
    return out


# speedup vs baseline: 41.4725x; 1.0605x over previous
import functools

import jax
import jax.numpy as jnp
from jax import lax
from jax.experimental import pallas as pl
from jax.experimental.pallas import tpu as pltpu
from jax.experimental.pallas import tpu_sc as plsc

N = 10000
D = 128
NC = 2
NS = 16
NW = NC * NS
K = 64
NBUF = 4
NIDX = 8
SROWS = 10240
RPT = SROWS // NS
RB = 2000
NB = N // RB

_f32 = jnp.float32
_sc_mesh = plsc.VectorSubcoreMesh(core_axis_name="c", subcore_axis_name="s")


def _make_deg_kernel(ept, e):
    @functools.partial(
        pl.kernel,
        out_type=jax.ShapeDtypeStruct((NC, SROWS), _f32),
        mesh=_sc_mesh,
        compiler_params=pltpu.CompilerParams(needs_layout_passes=False),
        scratch_types=[
            pltpu.VMEM((ept,), jnp.int32),
            pltpu.VMEM((SROWS,), _f32),
            pltpu.VMEM((NS, RPT), _f32),
            pltpu.VMEM((RPT,), _f32),
            pltpu.VMEM_SHARED((NS, SROWS), _f32),
        ],
    )
    def _deg_kernel(ei_hbm, out_hbm, dst_v, hist_v, red_v, out_v, sdeg):
        cid = lax.axis_index("c")
        sid = lax.axis_index("s")
        w = cid * NS + sid
        pltpu.sync_copy(ei_hbm.at[pl.ds(e + w * ept, ept)], dst_v)
        z16 = jnp.zeros((16,), _f32)
        o16 = jnp.ones((16,), _f32)

        def zb(t, c):
            hist_v[pl.ds(t * 16, 16)] = z16
            return c

        lax.fori_loop(0, SROWS // 16, zb, 0)

        def ab(t, c):
            idx = dst_v[pl.ds(t * 16, 16)]
            plsc.addupdate_scatter(hist_v, [idx], o16)
            return c

        lax.fori_loop(0, ept // 16, ab, 0)
        pltpu.sync_copy(hist_v, sdeg.at[sid])
        plsc.subcore_barrier()
        pltpu.sync_copy(sdeg.at[:, pl.ds(sid * RPT, RPT)], red_v)

        def rb_(t, c):
            v = red_v[0, pl.ds(t * 16, 16)]
            for r in range(1, NS):
                v = v + red_v[r, pl.ds(t * 16, 16)]
            out_v[pl.ds(t * 16, 16)] = v
            return c

        lax.fori_loop(0, RPT // 16, rb_, 0)
        pltpu.sync_copy(out_v, out_hbm.at[cid, pl.ds(sid * RPT, RPT)])

    return _deg_kernel


def _make_seg_kernel(nch, tk, ept, e):
    @functools.partial(
        pl.kernel,
        out_type=[jax.ShapeDtypeStruct((SROWS, D), _f32),
                  jax.ShapeDtypeStruct((SROWS, D), _f32)],
        mesh=_sc_mesh,
        scratch_types=[
            pltpu.VMEM((NIDX, K), jnp.int32),
            pltpu.VMEM((NIDX, K), jnp.int32),
            pltpu.VMEM((NBUF, K, D), _f32),
            pltpu.VMEM((max(tk, 8),), jnp.int32),
            pltpu.VMEM((max(tk, 8),), jnp.int32),
            pltpu.VMEM((max(tk, 8), D), _f32),
            pltpu.VMEM_SHARED((SROWS, D), _f32),
            pltpu.SemaphoreType.DMA((NIDX,)),
            pltpu.SemaphoreType.DMA((NIDX,)),
            pltpu.SemaphoreType.DMA((NBUF,)),
            pltpu.SemaphoreType.DMA((NBUF,)),
        ],
    )
    def _seg_kernel(y_hbm, ei_hbm, out0_hbm, out1_hbm,
                    sbuf, dbuf, rbuf, tsb, tdb, trb, sacc, si, sj, sg, sc):
        cid = lax.axis_index("c")
        sid = lax.axis_index("s")
        w = cid * NS + sid
        base = w * ept
        z16 = jnp.zeros((16,), _f32)

        def zb(t, c):
            r = t // 8
            cc = t - r * 8
            rbuf[0, r, pl.ds(cc * 16, 16)] = z16
            return c

        lax.fori_loop(0, K * (D // 16), zb, 0)
        for k in range(RPT // K):
            pltpu.sync_copy(rbuf.at[0], sacc.at[pl.ds(sid * RPT + k * K, K)])
        plsc.subcore_barrier()

        def idx_issue(j, q):
            pltpu.async_copy(ei_hbm.at[pl.ds(base + j * K, K)],
                             sbuf.at[q], si.at[q])
            pltpu.async_copy(ei_hbm.at[pl.ds(e + base + j * K, K)],
                             dbuf.at[q], sj.at[q])

        def gather_issue(j, q, b):
            pltpu.async_copy(y_hbm.at[sbuf.at[q]], rbuf.at[b], sg.at[b])

        def gather_wait(q, b):
            pltpu.make_async_copy(y_hbm.at[sbuf.at[q]], rbuf.at[b],
                                  sg.at[b]).wait()

        def scatter_issue(q, b):
            pltpu.async_copy(rbuf.at[b], sacc.at[dbuf.at[q]], sc.at[b],
                             add=True)

        def scatter_wait(q, b):
            pltpu.make_async_copy(rbuf.at[b], sacc.at[dbuf.at[q]],
                                  sc.at[b]).wait()

        def idx_wait(q):
            pltpu.make_async_copy(ei_hbm.at[pl.ds(base, K)], sbuf.at[q],
                                  si.at[q]).wait()

        def didx_wait(q):
            pltpu.make_async_copy(ei_hbm.at[pl.ds(base, K)], dbuf.at[q],
                                  sj.at[q]).wait()

        for q in range(6):
            idx_issue(q, q)
        for b in range(2):
            idx_wait(b)
            gather_issue(b, b, b)

        def body(t, c):
            for k in range(NIDX):
                j = NIDX * t + k
                b = k % NBUF
                q2 = (k + 2) % NIDX
                b2 = (k + 2) % NBUF
                q6 = (k + 6) % NIDX

                @pl.when(j < nch)
                def _():
                    gather_wait(k, b)
                    didx_wait(k)
                    scatter_issue(k, b)

                    @pl.when(jnp.logical_and(j + 2 < nch, j >= 2))
                    def _():
                        scatter_wait(q2, b2)

                    @pl.when(j + 2 < nch)
                    def _():
                        idx_wait(q2)
                        gather_issue(j + 2, q2, b2)

                    @pl.when(j + 6 < nch)
                    def _():
                        idx_issue(j + 6, q6)
            return c

        lax.fori_loop(0, -(-nch // NIDX), body, 0)
        for b in range(NBUF):
            scatter_wait(0, b)
        if tk:
            pltpu.async_copy(ei_hbm.at[pl.ds(base + nch * K, tk)], tsb,
                             si.at[0])
            pltpu.async_copy(ei_hbm.at[pl.ds(e + base + nch * K, tk)], tdb,
                             sj.at[0])
            pltpu.make_async_copy(ei_hbm.at[pl.ds(base, tk)], tsb,
                                  si.at[0]).wait()
            pltpu.async_copy(y_hbm.at[tsb], trb, sg.at[0]).wait()
            pltpu.make_async_copy(ei_hbm.at[pl.ds(base, tk)], tdb,
                                  sj.at[0]).wait()
            pltpu.sync_copy(trb, sacc.at[tdb], add=True)
        plsc.subcore_barrier()

        @pl.when(cid == 0)
        def _():
            pltpu.sync_copy(sacc.at[pl.ds(sid * RPT, RPT)],
                            out0_hbm.at[pl.ds(sid * RPT, RPT)])

        @pl.when(cid == 1)
        def _():
            pltpu.sync_copy(sacc.at[pl.ds(sid * RPT, RPT)],
                            out1_hbm.at[pl.ds(sid * RPT, RPT)])

    return _seg_kernel


def _scale_body(x_ref, p0_ref, p1_ref, y_ref, invd_ref, selfw_ref):
    d = p0_ref[...] + p1_ref[...] + 1.0
    invd = lax.rsqrt(d)
    invd_ref[...] = invd
    selfw_ref[...] = 1.0 / d
    y_ref[...] = x_ref[...] * invd


def _mmn_body(s0_ref, s1_ref, x_ref, ci_ref, cs_ref, w_ref, b_ref,
              g_ref, be_ref, al_ref, out_ref, h_ref, m1_ref, m2_ref):
    i = pl.program_id(0)

    @pl.when(i < NB)
    def _():
        agg = (ci_ref[...] * (s0_ref[...] + s1_ref[...])
               + cs_ref[...] * x_ref[...])
        h = jnp.dot(agg, w_ref[...], preferred_element_type=_f32) + b_ref[...]
        h_ref[...] = h

        @pl.when(i == 0)
        def _():
            m1_ref[...] = jnp.zeros_like(m1_ref)
            m2_ref[...] = jnp.zeros_like(m2_ref)

        m1_ref[0:1, :] += jnp.sum(h, axis=0, keepdims=True)
        m2_ref[0:1, :] += jnp.sum(h * h, axis=0, keepdims=True)

    @pl.when(i >= NB)
    def _():
        inv_n = 1.0 / float(N)
        mean = m1_ref[0:1, :] * inv_n
        ex2 = m2_ref[0:1, :] * inv_n
        a = al_ref[...]
        var = ex2 - (2.0 * a - a * a) * mean * mean
        h = s0_ref[...]
        h_ref[...] = h
        out_ref[...] = (g_ref[...] * (h - a * mean)
                        * lax.rsqrt(var + 1e-5) + be_ref[...])


def kernel(x, edge_index, W, b, gamma, beta, alpha):
    e = edge_index.shape[1]
    if e % (NW * 16) == 0:
        ept = e // NW
        ei = edge_index.reshape(-1)
    else:
        ept = -(-e // (NW * 16)) * 16
        pad_ar = jnp.arange(NW * ept - e, dtype=jnp.int32)
        ei = jnp.concatenate(
            [edge_index,
             jnp.stack([pad_ar % N, N + pad_ar % (SROWS - N)])],
            axis=1).reshape(-1)
    nch = ept // K
    tk = ept - nch * K

    degp = _make_deg_kernel(ept, NW * ept)(ei)

    dp = degp.reshape(NC, SROWS, 1)
    y, invd_col, selfw_col = pl.pallas_call(
        _scale_body,
        grid=(NB,),
        in_specs=[pl.BlockSpec((RB, D), lambda i: (i, 0)),
                  pl.BlockSpec((RB, 1), lambda i: (i, 0)),
                  pl.BlockSpec((RB, 1), lambda i: (i, 0))],
        out_specs=[pl.BlockSpec((RB, D), lambda i: (i, 0)),
                   pl.BlockSpec((RB, 1), lambda i: (i, 0)),
                   pl.BlockSpec((RB, 1), lambda i: (i, 0))],
        out_shape=[jax.ShapeDtypeStruct((N, D), _f32),
                   jax.ShapeDtypeStruct((N, 1), _f32),
                   jax.ShapeDtypeStruct((N, 1), _f32)],
    )(x, dp[0], dp[1])

    s0a, s1a = _make_seg_kernel(nch, tk, ept, NW * ept)(y, ei)

    mod_map = lambda i: (i % NB, 0)
    min_map = lambda i: (jnp.minimum(i, NB - 1), 0)
    zero_map = lambda i: (0, 0)
    out, _h, _m1, _m2 = pl.pallas_call(
        _mmn_body,
        grid=(2 * NB,),
        in_specs=[pl.BlockSpec((RB, D), mod_map),
                  pl.BlockSpec((RB, D), min_map),
                  pl.BlockSpec((RB, D), min_map),
                  pl.BlockSpec((RB, 1), min_map),
                  pl.BlockSpec((RB, 1), min_map),
                  pl.BlockSpec((D, D), zero_map),
                  pl.BlockSpec((1, D), zero_map),
                  pl.BlockSpec((1, D), zero_map),
                  pl.BlockSpec((1, D), zero_map),
                  pl.BlockSpec((1, D), zero_map)],
        out_specs=[pl.BlockSpec((RB, D), lambda i: (jnp.maximum(i - NB, 0), 0)),
                   pl.BlockSpec((RB, D), mod_map),
                   pl.BlockSpec((8, D), zero_map),
                   pl.BlockSpec((8, D), zero_map)],
        out_shape=[jax.ShapeDtypeStruct((N, D), _f32),
                   jax.ShapeDtypeStruct((SROWS, D), _f32),
                   jax.ShapeDtypeStruct((8, D), _f32),
                   jax.ShapeDtypeStruct((8, D), _f32)],
        input_output_aliases={0: 1},
    )(s0a, s1a, x, invd_col, selfw_col, W, b.reshape(1, D),
      gamma.reshape(1, D), beta.reshape(1, D), alpha.reshape(1, D))
    return out


# speedup vs baseline: 41.8611x; 1.0094x over previous
import functools

import jax
import jax.numpy as jnp
from jax import lax
from jax.experimental import pallas as pl
from jax.experimental.pallas import tpu as pltpu
from jax.experimental.pallas import tpu_sc as plsc

N = 10000
D = 128
NC = 2
NS = 16
NW = NC * NS
K = 64
NBUF = 4
NIDX = 8
SROWS = 10240
RPT = SROWS // NS
RB = 2000
NB = N // RB

_f32 = jnp.float32
_sc_mesh = plsc.VectorSubcoreMesh(core_axis_name="c", subcore_axis_name="s")


def _make_deg_kernel(ept, e):
    @functools.partial(
        pl.kernel,
        out_type=jax.ShapeDtypeStruct((NC, SROWS), _f32),
        mesh=_sc_mesh,
        compiler_params=pltpu.CompilerParams(needs_layout_passes=False),
        scratch_types=[
            pltpu.VMEM((ept,), jnp.int32),
            pltpu.VMEM((SROWS,), _f32),
            pltpu.VMEM((NS, RPT), _f32),
            pltpu.VMEM((RPT,), _f32),
            pltpu.VMEM_SHARED((NS, SROWS), _f32),
        ],
    )
    def _deg_kernel(ei_hbm, out_hbm, dst_v, hist_v, red_v, out_v, sdeg):
        cid = lax.axis_index("c")
        sid = lax.axis_index("s")
        w = cid * NS + sid
        pltpu.sync_copy(ei_hbm.at[pl.ds(e + w * ept, ept)], dst_v)
        z16 = jnp.zeros((16,), _f32)
        o16 = jnp.ones((16,), _f32)

        def zb(t, c):
            hist_v[pl.ds(t * 16, 16)] = z16
            return c

        lax.fori_loop(0, SROWS // 16, zb, 0)

        def ab(t, c):
            idx = dst_v[pl.ds(t * 16, 16)]
            plsc.addupdate_scatter(hist_v, [idx], o16)
            return c

        lax.fori_loop(0, ept // 16, ab, 0)
        pltpu.sync_copy(hist_v, sdeg.at[sid])
        plsc.subcore_barrier()
        pltpu.sync_copy(sdeg.at[:, pl.ds(sid * RPT, RPT)], red_v)

        def rb_(t, c):
            v = red_v[0, pl.ds(t * 16, 16)]
            for r in range(1, NS):
                v = v + red_v[r, pl.ds(t * 16, 16)]
            out_v[pl.ds(t * 16, 16)] = v
            return c

        lax.fori_loop(0, RPT // 16, rb_, 0)
        pltpu.sync_copy(out_v, out_hbm.at[cid, pl.ds(sid * RPT, RPT)])

    return _deg_kernel


def _make_seg_kernel(nch, tk, ept, e):
    @functools.partial(
        pl.kernel,
        out_type=[jax.ShapeDtypeStruct((SROWS, D), _f32),
                  jax.ShapeDtypeStruct((SROWS, D), _f32)],
        mesh=_sc_mesh,
        scratch_types=[
            pltpu.VMEM((NIDX, K), jnp.int32),
            pltpu.VMEM((NIDX, K), jnp.int32),
            pltpu.VMEM((NBUF, K, D), _f32),
            pltpu.VMEM((max(tk, 8),), jnp.int32),
            pltpu.VMEM((max(tk, 8),), jnp.int32),
            pltpu.VMEM((max(tk, 8), D), _f32),
            pltpu.VMEM_SHARED((SROWS, D), _f32),
            pltpu.SemaphoreType.DMA((NIDX,)),
            pltpu.SemaphoreType.DMA((NIDX,)),
            pltpu.SemaphoreType.DMA((NBUF,)),
            pltpu.SemaphoreType.DMA((NBUF,)),
        ],
    )
    def _seg_kernel(y_hbm, ei_hbm, out0_hbm, out1_hbm,
                    sbuf, dbuf, rbuf, tsb, tdb, trb, sacc, si, sj, sg, sc):
        cid = lax.axis_index("c")
        sid = lax.axis_index("s")
        w = cid * NS + sid
        base = w * ept
        z16 = jnp.zeros((16,), _f32)

        def zb(t, c):
            r = t // 8
            cc = t - r * 8
            rbuf[0, r, pl.ds(cc * 16, 16)] = z16
            return c

        lax.fori_loop(0, K * (D // 16), zb, 0)
        for k in range(RPT // K):
            pltpu.sync_copy(rbuf.at[0], sacc.at[pl.ds(sid * RPT + k * K, K)])
        plsc.subcore_barrier()

        def idx_issue(j, q):
            pltpu.async_copy(ei_hbm.at[pl.ds(base + j * K, K)],
                             sbuf.at[q], si.at[q])
            pltpu.async_copy(ei_hbm.at[pl.ds(e + base + j * K, K)],
                             dbuf.at[q], sj.at[q])

        def gather_issue(j, q, b):
            pltpu.async_copy(y_hbm.at[sbuf.at[q]], rbuf.at[b], sg.at[b])

        def gather_wait(q, b):
            pltpu.make_async_copy(y_hbm.at[sbuf.at[q]], rbuf.at[b],
                                  sg.at[b]).wait()

        def scatter_issue(q, b):
            pltpu.async_copy(rbuf.at[b], sacc.at[dbuf.at[q]], sc.at[b],
                             add=True)

        def scatter_wait(q, b):
            pltpu.make_async_copy(rbuf.at[b], sacc.at[dbuf.at[q]],
                                  sc.at[b]).wait()

        def idx_wait(q):
            pltpu.make_async_copy(ei_hbm.at[pl.ds(base, K)], sbuf.at[q],
                                  si.at[q]).wait()

        def didx_wait(q):
            pltpu.make_async_copy(ei_hbm.at[pl.ds(base, K)], dbuf.at[q],
                                  sj.at[q]).wait()

        for q in range(6):
            idx_issue(q, q)
        for b in range(2):
            idx_wait(b)
            gather_issue(b, b, b)

        def body(t, c):
            for k in range(NIDX):
                j = NIDX * t + k
                b = k % NBUF
                q2 = (k + 2) % NIDX
                b2 = (k + 2) % NBUF
                q6 = (k + 6) % NIDX

                @pl.when(j < nch)
                def _():
                    gather_wait(k, b)
                    didx_wait(k)
                    scatter_issue(k, b)

                    @pl.when(jnp.logical_and(j + 2 < nch, j >= 2))
                    def _():
                        scatter_wait(q2, b2)

                    @pl.when(j + 2 < nch)
                    def _():
                        idx_wait(q2)
                        gather_issue(j + 2, q2, b2)

                    @pl.when(j + 6 < nch)
                    def _():
                        idx_issue(j + 6, q6)
            return c

        lax.fori_loop(0, -(-nch // NIDX), body, 0)
        for b in range(NBUF):
            scatter_wait(0, b)
        if tk:
            pltpu.async_copy(ei_hbm.at[pl.ds(base + nch * K, tk)], tsb,
                             si.at[0])
            pltpu.async_copy(ei_hbm.at[pl.ds(e + base + nch * K, tk)], tdb,
                             sj.at[0])
            pltpu.make_async_copy(ei_hbm.at[pl.ds(base, tk)], tsb,
                                  si.at[0]).wait()
            pltpu.async_copy(y_hbm.at[tsb], trb, sg.at[0]).wait()
            pltpu.make_async_copy(ei_hbm.at[pl.ds(base, tk)], tdb,
                                  sj.at[0]).wait()
            pltpu.sync_copy(trb, sacc.at[tdb], add=True)
        plsc.subcore_barrier()

        @pl.when(cid == 0)
        def _():
            pltpu.sync_copy(sacc.at[pl.ds(sid * RPT, RPT)],
                            out0_hbm.at[pl.ds(sid * RPT, RPT)])

        @pl.when(cid == 1)
        def _():
            pltpu.sync_copy(sacc.at[pl.ds(sid * RPT, RPT)],
                            out1_hbm.at[pl.ds(sid * RPT, RPT)])

    return _seg_kernel


def _scale_body(x_ref, p0_ref, p1_ref, y_ref, invd_ref, selfw_ref):
    d = p0_ref[0] + p1_ref[0] + 1.0
    invd = lax.rsqrt(d)
    invd_ref[...] = invd
    selfw_ref[...] = 1.0 / d
    y_ref[...] = x_ref[...] * invd


def _mmn_body(s0_ref, s1_ref, x_ref, ci_ref, cs_ref, w_ref, b_ref,
              g_ref, be_ref, al_ref, out_ref, h_ref, m1_ref, m2_ref):
    i = pl.program_id(0)

    @pl.when(i < NB)
    def _():
        agg = (ci_ref[...] * (s0_ref[...] + s1_ref[...])
               + cs_ref[...] * x_ref[...])
        h = jnp.dot(agg, w_ref[...], preferred_element_type=_f32) + b_ref[...]
        h_ref[...] = h

        @pl.when(i == 0)
        def _():
            m1_ref[...] = jnp.zeros_like(m1_ref)
            m2_ref[...] = jnp.zeros_like(m2_ref)

        m1_ref[0:1, :] += jnp.sum(h, axis=0, keepdims=True)
        m2_ref[0:1, :] += jnp.sum(h * h, axis=0, keepdims=True)

    @pl.when(i >= NB)
    def _():
        inv_n = 1.0 / float(N)
        mean = m1_ref[0:1, :] * inv_n
        ex2 = m2_ref[0:1, :] * inv_n
        a = al_ref[...]
        var = ex2 - (2.0 * a - a * a) * mean * mean
        h = s0_ref[...]
        h_ref[...] = h
        out_ref[...] = (g_ref[...] * (h - a * mean)
                        * lax.rsqrt(var + 1e-5) + be_ref[...])


def kernel(x, edge_index, W, b, gamma, beta, alpha):
    e = edge_index.shape[1]
    if e % (NW * 16) == 0:
        ept = e // NW
        ei = edge_index.reshape(-1)
    else:
        ept = -(-e // (NW * 16)) * 16
        pad_ar = jnp.arange(NW * ept - e, dtype=jnp.int32)
        ei = jnp.concatenate(
            [edge_index,
             jnp.stack([pad_ar % N, N + pad_ar % (SROWS - N)])],
            axis=1).reshape(-1)
    nch = ept // K
    tk = ept - nch * K

    degp = _make_deg_kernel(ept, NW * ept)(ei)

    dp = degp.reshape(NC, SROWS, 1)
    y, invd_col, selfw_col = pl.pallas_call(
        _scale_body,
        grid=(NB,),
        in_specs=[pl.BlockSpec((RB, D), lambda i: (i, 0)),
                  pl.BlockSpec((1, RB, 1), lambda i: (0, i, 0)),
                  pl.BlockSpec((1, RB, 1), lambda i: (1, i, 0))],
        out_specs=[pl.BlockSpec((RB, D), lambda i: (i, 0)),
                   pl.BlockSpec((RB, 1), lambda i: (i, 0)),
                   pl.BlockSpec((RB, 1), lambda i: (i, 0))],
        out_shape=[jax.ShapeDtypeStruct((N, D), _f32),
                   jax.ShapeDtypeStruct((N, 1), _f32),
                   jax.ShapeDtypeStruct((N, 1), _f32)],
    )(x, dp, dp)

    s0a, s1a = _make_seg_kernel(nch, tk, ept, NW * ept)(y, ei)

    mod_map = lambda i: (i % NB, 0)
    min_map = lambda i: (jnp.minimum(i, NB - 1), 0)
    zero_map = lambda i: (0, 0)
    out, _h, _m1, _m2 = pl.pallas_call(
        _mmn_body,
        grid=(2 * NB,),
        in_specs=[pl.BlockSpec((RB, D), mod_map),
                  pl.BlockSpec((RB, D), min_map),
                  pl.BlockSpec((RB, D), min_map),
                  pl.BlockSpec((RB, 1), min_map),
                  pl.BlockSpec((RB, 1), min_map),
                  pl.BlockSpec((D, D), zero_map),
                  pl.BlockSpec((1, D), zero_map),
                  pl.BlockSpec((1, D), zero_map),
                  pl.BlockSpec((1, D), zero_map),
                  pl.BlockSpec((1, D), zero_map)],
        out_specs=[pl.BlockSpec((RB, D), lambda i: (jnp.maximum(i - NB, 0), 0)),
                   pl.BlockSpec((RB, D), mod_map),
                   pl.BlockSpec((8, D), zero_map),
                   pl.BlockSpec((8, D), zero_map)],
        out_shape=[jax.ShapeDtypeStruct((N, D), _f32),
                   jax.ShapeDtypeStruct((SROWS, D), _f32),
                   jax.ShapeDtypeStruct((8, D), _f32),
                   jax.ShapeDtypeStruct((8, D), _f32)],
        input_output_aliases={0: 1},
    )(s0a, s1a, x, invd_col, selfw_col, W, b.reshape(1, D),
      gamma.reshape(1, D), beta.reshape(1, D), alpha.reshape(1, D))
    return out


# speedup vs baseline: 42.7422x; 1.0210x over previous
import functools

import jax
import jax.numpy as jnp
from jax import lax
from jax.experimental import pallas as pl
from jax.experimental.pallas import tpu as pltpu
from jax.experimental.pallas import tpu_sc as plsc

N = 10000
D = 128
NC = 2
NS = 16
NW = NC * NS
K = 64
NBUF = 4
NIDX = 8
SROWS = 10240
RPT = SROWS // NS
RB = 2000
NB = N // RB

_f32 = jnp.float32
_sc_mesh = plsc.VectorSubcoreMesh(core_axis_name="c", subcore_axis_name="s")


def _make_deg_kernel(ept, e):
    @functools.partial(
        pl.kernel,
        out_type=jax.ShapeDtypeStruct((NC, SROWS), _f32),
        mesh=_sc_mesh,
        compiler_params=pltpu.CompilerParams(needs_layout_passes=False),
        scratch_types=[
            pltpu.VMEM((ept,), jnp.int32),
            pltpu.VMEM((SROWS,), _f32),
            pltpu.VMEM((NS, RPT), _f32),
            pltpu.VMEM((RPT,), _f32),
            pltpu.VMEM_SHARED((NS, SROWS), _f32),
        ],
    )
    def _deg_kernel(ei_hbm, out_hbm, dst_v, hist_v, red_v, out_v, sdeg):
        cid = lax.axis_index("c")
        sid = lax.axis_index("s")
        w = cid * NS + sid
        pltpu.sync_copy(ei_hbm.at[pl.ds(e + w * ept, ept)], dst_v)
        z16 = jnp.zeros((16,), _f32)
        o16 = jnp.ones((16,), _f32)

        def zb(t, c):
            hist_v[pl.ds(t * 64, 16)] = z16
            hist_v[pl.ds(t * 64 + 16, 16)] = z16
            hist_v[pl.ds(t * 64 + 32, 16)] = z16
            hist_v[pl.ds(t * 64 + 48, 16)] = z16
            return c

        lax.fori_loop(0, SROWS // 64, zb, 0)

        def ab(t, c):
            idx0 = dst_v[pl.ds(t * 64, 16)]
            idx1 = dst_v[pl.ds(t * 64 + 16, 16)]
            idx2 = dst_v[pl.ds(t * 64 + 32, 16)]
            idx3 = dst_v[pl.ds(t * 64 + 48, 16)]
            plsc.addupdate_scatter(hist_v, [idx0], o16)
            plsc.addupdate_scatter(hist_v, [idx1], o16)
            plsc.addupdate_scatter(hist_v, [idx2], o16)
            plsc.addupdate_scatter(hist_v, [idx3], o16)
            return c

        lax.fori_loop(0, ept // 64, ab, 0)
        for t in range(ept // 16 - (ept // 64) * 4):
            idx = dst_v[pl.ds((ept // 64) * 64 + t * 16, 16)]
            plsc.addupdate_scatter(hist_v, [idx], o16)
        pltpu.sync_copy(hist_v, sdeg.at[sid])
        plsc.subcore_barrier()
        pltpu.sync_copy(sdeg.at[:, pl.ds(sid * RPT, RPT)], red_v)

        def rb_(t, c):
            v = red_v[0, pl.ds(t * 16, 16)]
            for r in range(1, NS):
                v = v + red_v[r, pl.ds(t * 16, 16)]
            out_v[pl.ds(t * 16, 16)] = v
            return c

        lax.fori_loop(0, RPT // 16, rb_, 0)
        pltpu.sync_copy(out_v, out_hbm.at[cid, pl.ds(sid * RPT, RPT)])

    return _deg_kernel


def _make_seg_kernel(nch, tk, ept, e):
    @functools.partial(
        pl.kernel,
        out_type=[jax.ShapeDtypeStruct((SROWS, D), _f32),
                  jax.ShapeDtypeStruct((SROWS, D), _f32)],
        mesh=_sc_mesh,
        scratch_types=[
            pltpu.VMEM((NIDX, K), jnp.int32),
            pltpu.VMEM((NIDX, K), jnp.int32),
            pltpu.VMEM((NBUF, K, D), _f32),
            pltpu.VMEM((max(tk, 8),), jnp.int32),
            pltpu.VMEM((max(tk, 8),), jnp.int32),
            pltpu.VMEM((max(tk, 8), D), _f32),
            pltpu.VMEM_SHARED((SROWS, D), _f32),
            pltpu.SemaphoreType.DMA((NIDX,)),
            pltpu.SemaphoreType.DMA((NIDX,)),
            pltpu.SemaphoreType.DMA((NBUF,)),
            pltpu.SemaphoreType.DMA((NBUF,)),
        ],
    )
    def _seg_kernel(y_hbm, ei_hbm, out0_hbm, out1_hbm,
                    sbuf, dbuf, rbuf, tsb, tdb, trb, sacc, si, sj, sg, sc):
        cid = lax.axis_index("c")
        sid = lax.axis_index("s")
        w = cid * NS + sid
        base = w * ept
        z16 = jnp.zeros((16,), _f32)

        def zb(t, c):
            r = t // 8
            cc = t - r * 8
            rbuf[0, r, pl.ds(cc * 16, 16)] = z16
            return c

        lax.fori_loop(0, K * (D // 16), zb, 0)
        for k in range(RPT // K):
            pltpu.sync_copy(rbuf.at[0], sacc.at[pl.ds(sid * RPT + k * K, K)])
        plsc.subcore_barrier()

        def idx_issue(j, q):
            pltpu.async_copy(ei_hbm.at[pl.ds(base + j * K, K)],
                             sbuf.at[q], si.at[q])
            pltpu.async_copy(ei_hbm.at[pl.ds(e + base + j * K, K)],
                             dbuf.at[q], sj.at[q])

        def gather_issue(j, q, b):
            pltpu.async_copy(y_hbm.at[sbuf.at[q]], rbuf.at[b], sg.at[b])

        def gather_wait(q, b):
            pltpu.make_async_copy(y_hbm.at[sbuf.at[q]], rbuf.at[b],
                                  sg.at[b]).wait()

        def scatter_issue(q, b):
            pltpu.async_copy(rbuf.at[b], sacc.at[dbuf.at[q]], sc.at[b],
                             add=True)

        def scatter_wait(q, b):
            pltpu.make_async_copy(rbuf.at[b], sacc.at[dbuf.at[q]],
                                  sc.at[b]).wait()

        def idx_wait(q):
            pltpu.make_async_copy(ei_hbm.at[pl.ds(base, K)], sbuf.at[q],
                                  si.at[q]).wait()

        def didx_wait(q):
            pltpu.make_async_copy(ei_hbm.at[pl.ds(base, K)], dbuf.at[q],
                                  sj.at[q]).wait()

        for q in range(6):
            idx_issue(q, q)
        for b in range(2):
            idx_wait(b)
            gather_issue(b, b, b)

        def body(t, c):
            for k in range(NIDX):
                j = NIDX * t + k
                b = k % NBUF
                q2 = (k + 2) % NIDX
                b2 = (k + 2) % NBUF
                q6 = (k + 6) % NIDX

                @pl.when(j < nch)
                def _():
                    gather_wait(k, b)
                    didx_wait(k)
                    scatter_issue(k, b)

                    @pl.when(jnp.logical_and(j + 2 < nch, j >= 2))
                    def _():
                        scatter_wait(q2, b2)

                    @pl.when(j + 2 < nch)
                    def _():
                        idx_wait(q2)
                        gather_issue(j + 2, q2, b2)

                    @pl.when(j + 6 < nch)
                    def _():
                        idx_issue(j + 6, q6)
            return c

        lax.fori_loop(0, -(-nch // NIDX), body, 0)
        for b in range(NBUF):
            scatter_wait(0, b)
        if tk:
            pltpu.async_copy(ei_hbm.at[pl.ds(base + nch * K, tk)], tsb,
                             si.at[0])
            pltpu.async_copy(ei_hbm.at[pl.ds(e + base + nch * K, tk)], tdb,
                             sj.at[0])
            pltpu.make_async_copy(ei_hbm.at[pl.ds(base, tk)], tsb,
                                  si.at[0]).wait()
            pltpu.async_copy(y_hbm.at[tsb], trb, sg.at[0]).wait()
            pltpu.make_async_copy(ei_hbm.at[pl.ds(base, tk)], tdb,
                                  sj.at[0]).wait()
            pltpu.sync_copy(trb, sacc.at[tdb], add=True)
        plsc.subcore_barrier()

        @pl.when(cid == 0)
        def _():
            pltpu.sync_copy(sacc.at[pl.ds(sid * RPT, RPT)],
                            out0_hbm.at[pl.ds(sid * RPT, RPT)])

        @pl.when(cid == 1)
        def _():
            pltpu.sync_copy(sacc.at[pl.ds(sid * RPT, RPT)],
                            out1_hbm.at[pl.ds(sid * RPT, RPT)])

    return _seg_kernel


def _scale_body(x_ref, p0_ref, p1_ref, y_ref, invd_ref, selfw_ref):
    d = p0_ref[0] + p1_ref[0] + 1.0
    invd = lax.rsqrt(d)
    invd_ref[...] = invd
    selfw_ref[...] = 1.0 / d
    y_ref[...] = x_ref[...] * invd


def _mmn_body(s0_ref, s1_ref, x_ref, ci_ref, cs_ref, w_ref, b_ref,
              g_ref, be_ref, al_ref, out_ref, h_ref, m1_ref, m2_ref):
    i = pl.program_id(0)

    @pl.when(i < NB)
    def _():
        agg = (ci_ref[...] * (s0_ref[...] + s1_ref[...])
               + cs_ref[...] * x_ref[...])
        h = jnp.dot(agg, w_ref[...], preferred_element_type=_f32) + b_ref[...]
        h_ref[...] = h

        @pl.when(i == 0)
        def _():
            m1_ref[...] = jnp.zeros_like(m1_ref)
            m2_ref[...] = jnp.zeros_like(m2_ref)

        m1_ref[0:1, :] += jnp.sum(h, axis=0, keepdims=True)
        m2_ref[0:1, :] += jnp.sum(h * h, axis=0, keepdims=True)

    @pl.when(i >= NB)
    def _():
        inv_n = 1.0 / float(N)
        mean = m1_ref[0:1, :] * inv_n
        ex2 = m2_ref[0:1, :] * inv_n
        a = al_ref[...]
        var = ex2 - (2.0 * a - a * a) * mean * mean
        h = s0_ref[...]
        h_ref[...] = h
        out_ref[...] = (g_ref[...] * (h - a * mean)
                        * lax.rsqrt(var + 1e-5) + be_ref[...])


def kernel(x, edge_index, W, b, gamma, beta, alpha):
    e = edge_index.shape[1]
    if e % (NW * 16) == 0:
        ept = e // NW
        ei = edge_index.reshape(-1)
    else:
        ept = -(-e // (NW * 16)) * 16
        pad_ar = jnp.arange(NW * ept - e, dtype=jnp.int32)
        ei = jnp.concatenate(
            [edge_index,
             jnp.stack([pad_ar % N, N + pad_ar % (SROWS - N)])],
            axis=1).reshape(-1)
    nch = ept // K
    tk = ept - nch * K

    degp = _make_deg_kernel(ept, NW * ept)(ei)

    dp = degp.reshape(NC, SROWS, 1)
    y, invd_col, selfw_col = pl.pallas_call(
        _scale_body,
        grid=(NB,),
        in_specs=[pl.BlockSpec((RB, D), lambda i: (i, 0)),
                  pl.BlockSpec((1, RB, 1), lambda i: (0, i, 0)),
                  pl.BlockSpec((1, RB, 1), lambda i: (1, i, 0))],
        out_specs=[pl.BlockSpec((RB, D), lambda i: (i, 0)),
                   pl.BlockSpec((RB, 1), lambda i: (i, 0)),
                   pl.BlockSpec((RB, 1), lambda i: (i, 0))],
        out_shape=[jax.ShapeDtypeStruct((N, D), _f32),
                   jax.ShapeDtypeStruct((N, 1), _f32),
                   jax.ShapeDtypeStruct((N, 1), _f32)],
    )(x, dp, dp)

    s0a, s1a = _make_seg_kernel(nch, tk, ept, NW * ept)(y, ei)

    mod_map = lambda i: (i % NB, 0)
    min_map = lambda i: (jnp.minimum(i, NB - 1), 0)
    zero_map = lambda i: (0, 0)
    out, _h, _m1, _m2 = pl.pallas_call(
        _mmn_body,
        grid=(2 * NB,),
        in_specs=[pl.BlockSpec((RB, D), mod_map),
                  pl.BlockSpec((RB, D), min_map),
                  pl.BlockSpec((RB, D), min_map),
                  pl.BlockSpec((RB, 1), min_map),
                  pl.BlockSpec((RB, 1), min_map),
                  pl.BlockSpec((D, D), zero_map),
                  pl.BlockSpec((1, D), zero_map),
                  pl.BlockSpec((1, D), zero_map),
                  pl.BlockSpec((1, D), zero_map),
                  pl.BlockSpec((1, D), zero_map)],
        out_specs=[pl.BlockSpec((RB, D), lambda i: (jnp.maximum(i - NB, 0), 0)),
                   pl.BlockSpec((RB, D), mod_map),
                   pl.BlockSpec((8, D), zero_map),
                   pl.BlockSpec((8, D), zero_map)],
        out_shape=[jax.ShapeDtypeStruct((N, D), _f32),
                   jax.ShapeDtypeStruct((SROWS, D), _f32),
                   jax.ShapeDtypeStruct((8, D), _f32),
                   jax.ShapeDtypeStruct((8, D), _f32)],
        input_output_aliases={0: 1},
    )(s0a, s1a, x, invd_col, selfw_col, W, b.reshape(1, D),
      gamma.reshape(1, D), beta.reshape(1, D), alpha.reshape(1, D))
    return out


# speedup vs baseline: 42.7837x; 1.0010x over previous
import functools

import jax
import jax.numpy as jnp
from jax import lax
from jax.experimental import pallas as pl
from jax.experimental.pallas import tpu as pltpu
from jax.experimental.pallas import tpu_sc as plsc

N = 10000
D = 128
NC = 2
NS = 16
NW = NC * NS
K = 64
NBUF = 5
NIDX = 10
SROWS = 10240
RPT = SROWS // NS
RB = 2000
NB = N // RB

_f32 = jnp.float32
_sc_mesh = plsc.VectorSubcoreMesh(core_axis_name="c", subcore_axis_name="s")


def _make_deg_kernel(ept, e):
    @functools.partial(
        pl.kernel,
        out_type=jax.ShapeDtypeStruct((NC, SROWS), _f32),
        mesh=_sc_mesh,
        compiler_params=pltpu.CompilerParams(needs_layout_passes=False),
        scratch_types=[
            pltpu.VMEM((ept,), jnp.int32),
            pltpu.VMEM((SROWS,), _f32),
            pltpu.VMEM((NS, RPT), _f32),
            pltpu.VMEM((RPT,), _f32),
            pltpu.VMEM_SHARED((NS, SROWS), _f32),
        ],
    )
    def _deg_kernel(ei_hbm, out_hbm, dst_v, hist_v, red_v, out_v, sdeg):
        cid = lax.axis_index("c")
        sid = lax.axis_index("s")
        w = cid * NS + sid
        pltpu.sync_copy(ei_hbm.at[pl.ds(e + w * ept, ept)], dst_v)
        z16 = jnp.zeros((16,), _f32)
        o16 = jnp.ones((16,), _f32)

        def zb(t, c):
            hist_v[pl.ds(t * 64, 16)] = z16
            hist_v[pl.ds(t * 64 + 16, 16)] = z16
            hist_v[pl.ds(t * 64 + 32, 16)] = z16
            hist_v[pl.ds(t * 64 + 48, 16)] = z16
            return c

        lax.fori_loop(0, SROWS // 64, zb, 0)

        def ab(t, c):
            idx0 = dst_v[pl.ds(t * 64, 16)]
            idx1 = dst_v[pl.ds(t * 64 + 16, 16)]
            idx2 = dst_v[pl.ds(t * 64 + 32, 16)]
            idx3 = dst_v[pl.ds(t * 64 + 48, 16)]
            plsc.addupdate_scatter(hist_v, [idx0], o16)
            plsc.addupdate_scatter(hist_v, [idx1], o16)
            plsc.addupdate_scatter(hist_v, [idx2], o16)
            plsc.addupdate_scatter(hist_v, [idx3], o16)
            return c

        lax.fori_loop(0, ept // 64, ab, 0)
        for t in range(ept // 16 - (ept // 64) * 4):
            idx = dst_v[pl.ds((ept // 64) * 64 + t * 16, 16)]
            plsc.addupdate_scatter(hist_v, [idx], o16)
        pltpu.sync_copy(hist_v, sdeg.at[sid])
        plsc.subcore_barrier()
        pltpu.sync_copy(sdeg.at[:, pl.ds(sid * RPT, RPT)], red_v)

        def rb_(t, c):
            v = red_v[0, pl.ds(t * 16, 16)]
            for r in range(1, NS):
                v = v + red_v[r, pl.ds(t * 16, 16)]
            out_v[pl.ds(t * 16, 16)] = v
            return c

        lax.fori_loop(0, RPT // 16, rb_, 0)
        pltpu.sync_copy(out_v, out_hbm.at[cid, pl.ds(sid * RPT, RPT)])

    return _deg_kernel


def _make_seg_kernel(nch, tk, ept, e):
    @functools.partial(
        pl.kernel,
        out_type=[jax.ShapeDtypeStruct((SROWS, D), _f32),
                  jax.ShapeDtypeStruct((SROWS, D), _f32)],
        mesh=_sc_mesh,
        scratch_types=[
            pltpu.VMEM((NIDX, K), jnp.int32),
            pltpu.VMEM((NIDX, K), jnp.int32),
            pltpu.VMEM((NBUF, K, D), _f32),
            pltpu.VMEM((max(tk, 8),), jnp.int32),
            pltpu.VMEM((max(tk, 8),), jnp.int32),
            pltpu.VMEM((max(tk, 8), D), _f32),
            pltpu.VMEM_SHARED((SROWS, D), _f32),
            pltpu.SemaphoreType.DMA((NIDX,)),
            pltpu.SemaphoreType.DMA((NIDX,)),
            pltpu.SemaphoreType.DMA((NBUF,)),
            pltpu.SemaphoreType.DMA((NBUF,)),
        ],
    )
    def _seg_kernel(y_hbm, ei_hbm, out0_hbm, out1_hbm,
                    sbuf, dbuf, rbuf, tsb, tdb, trb, sacc, si, sj, sg, sc):
        cid = lax.axis_index("c")
        sid = lax.axis_index("s")
        w = cid * NS + sid
        base = w * ept
        z16 = jnp.zeros((16,), _f32)

        def zb(t, c):
            r = t // 8
            cc = t - r * 8
            rbuf[0, r, pl.ds(cc * 16, 16)] = z16
            return c

        lax.fori_loop(0, K * (D // 16), zb, 0)
        for k in range(RPT // K):
            pltpu.sync_copy(rbuf.at[0], sacc.at[pl.ds(sid * RPT + k * K, K)])
        plsc.subcore_barrier()

        def idx_issue(j, q):
            pltpu.async_copy(ei_hbm.at[pl.ds(base + j * K, K)],
                             sbuf.at[q], si.at[q])
            pltpu.async_copy(ei_hbm.at[pl.ds(e + base + j * K, K)],
                             dbuf.at[q], sj.at[q])

        def gather_issue(j, q, b):
            pltpu.async_copy(y_hbm.at[sbuf.at[q]], rbuf.at[b], sg.at[b])

        def gather_wait(q, b):
            pltpu.make_async_copy(y_hbm.at[sbuf.at[q]], rbuf.at[b],
                                  sg.at[b]).wait()

        def scatter_issue(q, b):
            pltpu.async_copy(rbuf.at[b], sacc.at[dbuf.at[q]], sc.at[b],
                             add=True)

        def scatter_wait(q, b):
            pltpu.make_async_copy(rbuf.at[b], sacc.at[dbuf.at[q]],
                                  sc.at[b]).wait()

        def idx_wait(q):
            pltpu.make_async_copy(ei_hbm.at[pl.ds(base, K)], sbuf.at[q],
                                  si.at[q]).wait()

        def didx_wait(q):
            pltpu.make_async_copy(ei_hbm.at[pl.ds(base, K)], dbuf.at[q],
                                  sj.at[q]).wait()

        for q in range(6):
            idx_issue(q, q)
        for b in range(2):
            idx_wait(b)
            gather_issue(b, b, b)

        def body(t, c):
            for k in range(NIDX):
                j = NIDX * t + k
                b = k % NBUF
                q2 = (k + 2) % NIDX
                b2 = (k + 2) % NBUF
                q6 = (k + 6) % NIDX

                @pl.when(j < nch)
                def _():
                    gather_wait(k, b)
                    didx_wait(k)
                    scatter_issue(k, b)

                    @pl.when(jnp.logical_and(j + 2 < nch, j >= 3))
                    def _():
                        scatter_wait(q2, b2)

                    @pl.when(j + 2 < nch)
                    def _():
                        idx_wait(q2)
                        gather_issue(j + 2, q2, b2)

                    @pl.when(j + 6 < nch)
                    def _():
                        idx_issue(j + 6, q6)
            return c

        lax.fori_loop(0, -(-nch // NIDX), body, 0)
        for b in range(NBUF):
            scatter_wait(0, b)
        if tk:
            pltpu.async_copy(ei_hbm.at[pl.ds(base + nch * K, tk)], tsb,
                             si.at[0])
            pltpu.async_copy(ei_hbm.at[pl.ds(e + base + nch * K, tk)], tdb,
                             sj.at[0])
            pltpu.make_async_copy(ei_hbm.at[pl.ds(base, tk)], tsb,
                                  si.at[0]).wait()
            pltpu.async_copy(y_hbm.at[tsb], trb, sg.at[0]).wait()
            pltpu.make_async_copy(ei_hbm.at[pl.ds(base, tk)], tdb,
                                  sj.at[0]).wait()
            pltpu.sync_copy(trb, sacc.at[tdb], add=True)
        plsc.subcore_barrier()

        @pl.when(cid == 0)
        def _():
            pltpu.sync_copy(sacc.at[pl.ds(sid * RPT, RPT)],
                            out0_hbm.at[pl.ds(sid * RPT, RPT)])

        @pl.when(cid == 1)
        def _():
            pltpu.sync_copy(sacc.at[pl.ds(sid * RPT, RPT)],
                            out1_hbm.at[pl.ds(sid * RPT, RPT)])

    return _seg_kernel


def _scale_body(x_ref, p0_ref, p1_ref, y_ref, invd_ref, selfw_ref):
    d = p0_ref[0] + p1_ref[0] + 1.0
    invd = lax.rsqrt(d)
    invd_ref[...] = invd
    selfw_ref[...] = 1.0 / d
    y_ref[...] = x_ref[...] * invd


def _mmn_body(s0_ref, s1_ref, x_ref, ci_ref, cs_ref, w_ref, b_ref,
              g_ref, be_ref, al_ref, out_ref, h_ref, m1_ref, m2_ref):
    i = pl.program_id(0)

    @pl.when(i < NB)
    def _():
        agg = (ci_ref[...] * (s0_ref[...] + s1_ref[...])
               + cs_ref[...] * x_ref[...])
        h = jnp.dot(agg, w_ref[...], preferred_element_type=_f32) + b_ref[...]
        h_ref[...] = h

        @pl.when(i == 0)
        def _():
            m1_ref[...] = jnp.zeros_like(m1_ref)
            m2_ref[...] = jnp.zeros_like(m2_ref)

        m1_ref[0:1, :] += jnp.sum(h, axis=0, keepdims=True)
        m2_ref[0:1, :] += jnp.sum(h * h, axis=0, keepdims=True)

    @pl.when(i >= NB)
    def _():
        inv_n = 1.0 / float(N)
        mean = m1_ref[0:1, :] * inv_n
        ex2 = m2_ref[0:1, :] * inv_n
        a = al_ref[...]
        var = ex2 - (2.0 * a - a * a) * mean * mean
        h = s0_ref[...]
        h_ref[...] = h
        out_ref[...] = (g_ref[...] * (h - a * mean)
                        * lax.rsqrt(var + 1e-5) + be_ref[...])


def kernel(x, edge_index, W, b, gamma, beta, alpha):
    e = edge_index.shape[1]
    if e % (NW * 16) == 0:
        ept = e // NW
        ei = edge_index.reshape(-1)
    else:
        ept = -(-e // (NW * 16)) * 16
        pad_ar = jnp.arange(NW * ept - e, dtype=jnp.int32)
        ei = jnp.concatenate(
            [edge_index,
             jnp.stack([pad_ar % N, N + pad_ar % (SROWS - N)])],
            axis=1).reshape(-1)
    nch = ept // K
    tk = ept - nch * K

    degp = _make_deg_kernel(ept, NW * ept)(ei)

    dp = degp.reshape(NC, SROWS, 1)
    y, invd_col, selfw_col = pl.pallas_call(
        _scale_body,
        grid=(NB,),
        in_specs=[pl.BlockSpec((RB, D), lambda i: (i, 0)),
                  pl.BlockSpec((1, RB, 1), lambda i: (0, i, 0)),
                  pl.BlockSpec((1, RB, 1), lambda i: (1, i, 0))],
        out_specs=[pl.BlockSpec((RB, D), lambda i: (i, 0)),
                   pl.BlockSpec((RB, 1), lambda i: (i, 0)),
                   pl.BlockSpec((RB, 1), lambda i: (i, 0))],
        out_shape=[jax.ShapeDtypeStruct((N, D), _f32),
                   jax.ShapeDtypeStruct((N, 1), _f32),
                   jax.ShapeDtypeStruct((N, 1), _f32)],
    )(x, dp, dp)

    s0a, s1a = _make_seg_kernel(nch, tk, ept, NW * ept)(y, ei)

    mod_map = lambda i: (i % NB, 0)
    min_map = lambda i: (jnp.minimum(i, NB - 1), 0)
    zero_map = lambda i: (0, 0)
    out, _h, _m1, _m2 = pl.pallas_call(
        _mmn_body,
        grid=(2 * NB,),
        in_specs=[pl.BlockSpec((RB, D), mod_map),
                  pl.BlockSpec((RB, D), min_map),
                  pl.BlockSpec((RB, D), min_map),
                  pl.BlockSpec((RB, 1), min_map),
                  pl.BlockSpec((RB, 1), min_map),
                  pl.BlockSpec((D, D), zero_map),
                  pl.BlockSpec((1, D), zero_map),
                  pl.BlockSpec((1, D), zero_map),
                  pl.BlockSpec((1, D), zero_map),
                  pl.BlockSpec((1, D), zero_map)],
        out_specs=[pl.BlockSpec((RB, D), lambda i: (jnp.maximum(i - NB, 0), 0)),
                   pl.BlockSpec((RB, D), mod_map),
                   pl.BlockSpec((8, D), zero_map),
                   pl.BlockSpec((8, D), zero_map)],
        out_shape=[jax.ShapeDtypeStruct((N, D), _f32),
                   jax.ShapeDtypeStruct((SROWS, D), _f32),
                   jax.ShapeDtypeStruct((8, D), _f32),
                   jax.ShapeDtypeStruct((8, D), _f32)],
        input_output_aliases={0: 1},
    )(s0a, s1a, x, invd_col, selfw_col, W, b.reshape(1, D),
      gamma.reshape(1, D), beta.reshape(1, D), alpha.reshape(1, D))
    return out
